# baseline (speedup 1.0000x reference)
import functools

import jax
import jax.numpy as jnp
from jax import lax
from jax.experimental import pallas as pl
from jax.experimental.pallas import tpu as pltpu
from jax.experimental.pallas import tpu_sc as plsc

N = 10000
E = 320000
D = 128
NP_ = 10240
NC, NS, L = 2, 16, 16
NW = NC * NS
CPT = D // NW
EPW = 10240
EP = EPW * NW
CH = 128
NCH = EPW // CH
BE = 4096
NBE = EP // BE
RPT = NP_ // NS

_f32 = jnp.float32
_params = pltpu.CompilerParams(use_tc_tiling_on_sc=False,
                               needs_layout_passes=False)


def _wid():
    c = lax.axis_index("c")
    s = lax.axis_index("s")
    return c, s, c * NS + s


@functools.cache
def _sc_kernels():
    mesh = plsc.VectorSubcoreMesh(core_axis_name="c", subcore_axis_name="s",
                                  num_cores=NC, num_subcores=NS)

    def _reduce_tiles(tab_v, red_v, stage_sh, out_ref, c, s):
        pltpu.sync_copy(tab_v, stage_sh.at[s])
        plsc.subcore_barrier()
        for t in range(NS):
            pltpu.sync_copy(stage_sh.at[t, pl.ds(s * RPT, RPT)], red_v)
            if t == 0:
                def cp0(k, carry):
                    tab_v[pl.ds(k * L, L)] = red_v[pl.ds(k * L, L)]
                    return carry
                lax.fori_loop(0, RPT // L, cp0, 0)
            else:
                def acc_t(k, carry):
                    tab_v[pl.ds(k * L, L)] = (tab_v[pl.ds(k * L, L)]
                                              + red_v[pl.ds(k * L, L)])
                    return carry
                lax.fori_loop(0, RPT // L, acc_t, 0)
        pltpu.sync_copy(tab_v.at[pl.ds(0, RPT)],
                        out_ref.at[c, pl.ds(s * RPT, RPT)])

    @functools.partial(
        pl.kernel,
        out_type=jax.ShapeDtypeStruct((NC, NP_), _f32),
        mesh=mesh,
        compiler_params=_params,
        scratch_types=[
            pltpu.VMEM((NCH, CH), jnp.int32),
            pltpu.VMEM((NP_,), _f32),
            pltpu.VMEM((RPT,), _f32),
            pltpu.VMEM_SHARED((NS, NP_), _f32),
        ],
    )
    def _sc_deg(dst_hbm, z1_hbm, deg_out, dst_v, tab_v, red_v, stage_sh):
        c, s, w = _wid()
        pltpu.sync_copy(dst_hbm.at[w], dst_v)
        pltpu.sync_copy(z1_hbm, tab_v)
        ones = jnp.ones((L,), _f32)

        def chunk(j, carry):
            for k in range(CH // L):
                dv = dst_v[j, pl.ds(k * L, L)]
                plsc.addupdate_scatter(tab_v, [dv], ones)
            return carry

        lax.fori_loop(0, NCH, chunk, 0)
        _reduce_tiles(tab_v, red_v, stage_sh, deg_out, c, s)

    @functools.partial(
        pl.kernel,
        out_type=jax.ShapeDtypeStruct((D * NP_,), _f32),
        mesh=mesh,
        compiler_params=_params,
        scratch_types=[
            pltpu.VMEM((CPT * NP_,), _f32),
            pltpu.VMEM((CPT * NP_,), _f32),
            pltpu.VMEM((BE,), jnp.int32),
            pltpu.VMEM((BE,), jnp.int32),
            pltpu.VMEM((BE,), jnp.int32),
            pltpu.VMEM((BE,), jnp.int32),
            pltpu.SemaphoreType.DMA,
            pltpu.SemaphoreType.DMA,
        ],
    )
    def _sc_agg(yt_hbm, src_hbm, dst_hbm, out_hbm,
                tab_v, acc_v, sb0, db0, sb1, db1, i0, i1):
        c, s, w = _wid()
        pltpu.sync_copy(yt_hbm.at[pl.ds(w * CPT * NP_, CPT * NP_)], tab_v)

        def zero(k, carry):
            acc_v[pl.ds(k * L, L)] = jnp.zeros((L,), _f32)
            return carry

        lax.fori_loop(0, CPT * NP_ // L, zero, 0)
        off = [jnp.full((L,), cc * NP_, jnp.int32) for cc in range(CPT)]
        UN = 4

        def process(sb, db):
            def grp(g, carry):
                for uu in range(UN):
                    srcv = sb[pl.ds((g * UN + uu) * L, L)]
                    dstv = db[pl.ds((g * UN + uu) * L, L)]
                    for cc in range(CPT):
                        v = plsc.load_gather(tab_v, [srcv + off[cc]])
                        plsc.addupdate_scatter(acc_v, [dstv + off[cc]], v)
                return carry

            lax.fori_loop(0, BE // L // UN, grp, 0)

        pltpu.async_copy(src_hbm.at[pl.ds(0, BE)], sb0, i0)
        pltpu.async_copy(dst_hbm.at[pl.ds(0, BE)], db0, i0)
        pltpu.async_copy(src_hbm.at[pl.ds(BE, BE)], sb1, i1)
        pltpu.async_copy(dst_hbm.at[pl.ds(BE, BE)], db1, i1)

        def pair(i, carry):
            a = 2 * i
            b = a + 1
            pltpu.make_async_copy(src_hbm.at[pl.ds(0, BE)], sb0, i0).wait()
            pltpu.make_async_copy(dst_hbm.at[pl.ds(0, BE)], db0, i0).wait()
            process(sb0, db0)

            @pl.when(i < NBE // 2 - 1)
            def _():
                pltpu.async_copy(src_hbm.at[pl.ds((a + 2) * BE, BE)],
                                 sb0, i0)
                pltpu.async_copy(dst_hbm.at[pl.ds((a + 2) * BE, BE)],
                                 db0, i0)

            pltpu.make_async_copy(src_hbm.at[pl.ds(0, BE)], sb1, i1).wait()
            pltpu.make_async_copy(dst_hbm.at[pl.ds(0, BE)], db1, i1).wait()
            process(sb1, db1)

            @pl.when(i < NBE // 2 - 1)
            def _():
                pltpu.async_copy(src_hbm.at[pl.ds((b + 2) * BE, BE)],
                                 sb1, i1)
                pltpu.async_copy(dst_hbm.at[pl.ds((b + 2) * BE, BE)],
                                 db1, i1)

            return carry

        lax.fori_loop(0, NBE // 2, pair, 0)
        pltpu.sync_copy(acc_v, out_hbm.at[pl.ds(w * CPT * NP_, CPT * NP_)])

    @functools.partial(
        pl.kernel,
        out_type=(jax.ShapeDtypeStruct((NW, NCH, CH), _f32),
                  jax.ShapeDtypeStruct((NC, NP_), _f32)),
        mesh=mesh,
        compiler_params=_params,
        scratch_types=[
            pltpu.VMEM((NCH, CH), jnp.int32),
            pltpu.VMEM((NCH, CH), jnp.int32),
            pltpu.VMEM((CH, D), _f32),
            pltpu.VMEM((CH, D), _f32),
            pltpu.VMEM((CH, D), _f32),
            pltpu.VMEM((CH, D), _f32),
            pltpu.VMEM((NCH, CH), _f32),
            pltpu.VMEM((D,), _f32),
            pltpu.VMEM((NP_,), _f32),
            pltpu.VMEM((RPT,), _f32),
            pltpu.VMEM_SHARED((NS, NP_), _f32),
            pltpu.SemaphoreType.DMA,
            pltpu.SemaphoreType.DMA,
            pltpu.SemaphoreType.DMA,
            pltpu.SemaphoreType.DMA,
        ],
    )
    def _sc_edge_e(fs_hbm, fd_hbm, src_hbm, dst_hbm, attn_hbm, z1_hbm,
                   e_out, se_out,
                   src_v, dst_v, fsb0, fsb1, fdb0, fdb1, e_vm, attn_v,
                   tab_v, red_v, stage_sh,
                   gs0, gs1, gd0, gd1):
        c, s, w = _wid()
        pltpu.sync_copy(src_hbm.at[w], src_v)
        pltpu.sync_copy(dst_hbm.at[w], dst_v)
        pltpu.sync_copy(attn_hbm, attn_v)
        pltpu.sync_copy(z1_hbm, tab_v)
        pltpu.async_copy(fs_hbm.at[src_v.at[0]], fsb0, gs0)
        pltpu.async_copy(fd_hbm.at[dst_v.at[0]], fdb0, gd0)
        pltpu.async_copy(fs_hbm.at[src_v.at[1]], fsb1, gs1)
        pltpu.async_copy(fd_hbm.at[dst_v.at[1]], fdb1, gd1)

        lane = lax.iota(jnp.int32, L)

        def compute_chunk(j, fsb, fdb):
            def grp16(g, carry):
                accs = []
                for rr in range(L):
                    r = g * L + rr
                    acc = jnp.zeros((L,), _f32)
                    for dg in range(D // L):
                        x = (fsb[r, pl.ds(dg * L, L)]
                             + fdb[r, pl.ds(dg * L, L)])
                        t = jnp.where(x > 0.0, x, 0.2 * x)
                        acc = acc + t * attn_v[pl.ds(dg * L, L)]
                    accs.append(acc)
                vacc = jnp.zeros((L,), _f32)
                for rr in range(L):
                    vacc = jnp.where(lane == rr, jnp.sum(accs[rr]), vacc)
                e_vm[j, pl.ds(g * L, L)] = vacc
                dv = dst_v[j, pl.ds(g * L, L)]
                plsc.addupdate_scatter(tab_v, [dv], vacc)
                return carry

            lax.fori_loop(0, CH // L, grp16, 0)

        def pair(jj, carry):
            a = 2 * jj
            b = a + 1
            pltpu.make_async_copy(fs_hbm.at[src_v.at[a]], fsb0, gs0).wait()
            pltpu.make_async_copy(fd_hbm.at[dst_v.at[a]], fdb0, gd0).wait()
            compute_chunk(a, fsb0, fdb0)

            @pl.when(jj < NCH // 2 - 1)
            def _():
                pltpu.async_copy(fs_hbm.at[src_v.at[a + 2]], fsb0, gs0)
                pltpu.async_copy(fd_hbm.at[dst_v.at[a + 2]], fdb0, gd0)

            pltpu.make_async_copy(fs_hbm.at[src_v.at[b]], fsb1, gs1).wait()
            pltpu.make_async_copy(fd_hbm.at[dst_v.at[b]], fdb1, gd1).wait()
            compute_chunk(b, fsb1, fdb1)

            @pl.when(jj < NCH // 2 - 1)
            def _():
                pltpu.async_copy(fs_hbm.at[src_v.at[b + 2]], fsb1, gs1)
                pltpu.async_copy(fd_hbm.at[dst_v.at[b + 2]], fdb1, gd1)

            return carry

        lax.fori_loop(0, NCH // 2, pair, 0)
        pltpu.sync_copy(e_vm, e_out.at[w])
        _reduce_tiles(tab_v, red_v, stage_sh, se_out, c, s)

    @functools.partial(
        pl.kernel,
        out_type=(jax.ShapeDtypeStruct((NW, NCH, CH), _f32),
                  jax.ShapeDtypeStruct((NC, NP_), _f32)),
        mesh=mesh,
        compiler_params=_params,
        scratch_types=[
            pltpu.VMEM((NCH, CH), jnp.int32),
            pltpu.VMEM((NCH, CH), _f32),
            pltpu.VMEM((NCH, CH), _f32),
            pltpu.VMEM((NP_,), _f32),
            pltpu.VMEM((NP_,), _f32),
            pltpu.VMEM((RPT,), _f32),
            pltpu.VMEM_SHARED((NS, NP_), _f32),
        ],
    )
    def _sc_softmax_num(e_hbm, dst_hbm, se_hbm, degc_hbm, z1_hbm,
                        ee_out, s_out,
                        dst_v, e_vm, ee_vm, b_tab, tab_v, red_v, stage_sh):
        c, s, w = _wid()
        pltpu.sync_copy(dst_hbm.at[w], dst_v)
        pltpu.sync_copy(e_hbm.at[w], e_vm)
        pltpu.sync_copy(se_hbm.at[0], b_tab)
        pltpu.sync_copy(se_hbm.at[1], tab_v)

        def add_grp(k, carry):
            b_tab[pl.ds(k * L, L)] = (b_tab[pl.ds(k * L, L)]
                                      + tab_v[pl.ds(k * L, L)])
            return carry

        lax.fori_loop(0, NP_ // L, add_grp, 0)
        pltpu.sync_copy(degc_hbm, tab_v)

        def div_grp(k, carry):
            b_tab[pl.ds(k * L, L)] = (b_tab[pl.ds(k * L, L)]
                                      / tab_v[pl.ds(k * L, L)])
            return carry

        lax.fori_loop(0, NP_ // L, div_grp, 0)
        pltpu.sync_copy(z1_hbm, tab_v)

        def chunk(j, carry):
            def grp(k, carry2):
                dv = dst_v[j, pl.ds(k * L, L)]
                bv = plsc.load_gather(b_tab, [dv])
                ee = jnp.exp(e_vm[j, pl.ds(k * L, L)] - bv)
                ee_vm[j, pl.ds(k * L, L)] = ee
                plsc.addupdate_scatter(tab_v, [dv], ee)
                return carry2

            lax.fori_loop(0, CH // L, grp, 0)
            return carry

        lax.fori_loop(0, NCH, chunk, 0)
        pltpu.sync_copy(ee_vm, ee_out.at[w])
        _reduce_tiles(tab_v, red_v, stage_sh, s_out, c, s)

    @functools.partial(
        pl.kernel,
        out_type=jax.ShapeDtypeStruct((D * NP_,), _f32),
        mesh=mesh,
        compiler_params=_params,
        scratch_types=[
            pltpu.VMEM((CPT * NP_,), _f32),
            pltpu.VMEM((CPT * NP_,), _f32),
            pltpu.VMEM((BE,), jnp.int32),
            pltpu.VMEM((BE,), jnp.int32),
            pltpu.VMEM((BE,), _f32),
            pltpu.VMEM((BE,), jnp.int32),
            pltpu.VMEM((BE,), jnp.int32),
            pltpu.VMEM((BE,), _f32),
            pltpu.SemaphoreType.DMA,
            pltpu.SemaphoreType.DMA,
        ],
    )
    def _sc_wagg(fst_hbm, src_hbm, dst_hbm, ee_hbm, out_hbm,
                 tab_v, acc_v, sb0, db0, eb0, sb1, db1, eb1, i0, i1):
        c, s, w = _wid()
        pltpu.sync_copy(fst_hbm.at[pl.ds(w * CPT * NP_, CPT * NP_)], tab_v)

        def zero(k, carry):
            acc_v[pl.ds(k * L, L)] = jnp.zeros((L,), _f32)
            return carry

        lax.fori_loop(0, CPT * NP_ // L, zero, 0)
        off = [jnp.full((L,), cc * NP_, jnp.int32) for cc in range(CPT)]
        UN = 4

        def process(sb, db, eb):
            def grp(g, carry):
                for uu in range(UN):
                    srcv = sb[pl.ds((g * UN + uu) * L, L)]
                    dstv = db[pl.ds((g * UN + uu) * L, L)]
                    eev = eb[pl.ds((g * UN + uu) * L, L)]
                    for cc in range(CPT):
                        v = plsc.load_gather(tab_v, [srcv + off[cc]]) * eev
                        plsc.addupdate_scatter(acc_v, [dstv + off[cc]], v)
                return carry

            lax.fori_loop(0, BE // L // UN, grp, 0)

        pltpu.async_copy(src_hbm.at[pl.ds(0, BE)], sb0, i0)
        pltpu.async_copy(dst_hbm.at[pl.ds(0, BE)], db0, i0)
        pltpu.async_copy(ee_hbm.at[pl.ds(0, BE)], eb0, i0)
        pltpu.async_copy(src_hbm.at[pl.ds(BE, BE)], sb1, i1)
        pltpu.async_copy(dst_hbm.at[pl.ds(BE, BE)], db1, i1)
        pltpu.async_copy(ee_hbm.at[pl.ds(BE, BE)], eb1, i1)

        def pair(i, carry):
            a = 2 * i
            b = a + 1
            pltpu.make_async_copy(src_hbm.at[pl.ds(0, BE)], sb0, i0).wait()
            pltpu.make_async_copy(dst_hbm.at[pl.ds(0, BE)], db0, i0).wait()
            pltpu.make_async_copy(ee_hbm.at[pl.ds(0, BE)], eb0, i0).wait()
            process(sb0, db0, eb0)

            @pl.when(i < NBE // 2 - 1)
            def _():
                pltpu.async_copy(src_hbm.at[pl.ds((a + 2) * BE, BE)],
                                 sb0, i0)
                pltpu.async_copy(dst_hbm.at[pl.ds((a + 2) * BE, BE)],
                                 db0, i0)
                pltpu.async_copy(ee_hbm.at[pl.ds((a + 2) * BE, BE)],
                                 eb0, i0)

            pltpu.make_async_copy(src_hbm.at[pl.ds(0, BE)], sb1, i1).wait()
            pltpu.make_async_copy(dst_hbm.at[pl.ds(0, BE)], db1, i1).wait()
            pltpu.make_async_copy(ee_hbm.at[pl.ds(0, BE)], eb1, i1).wait()
            process(sb1, db1, eb1)

            @pl.when(i < NBE // 2 - 1)
            def _():
                pltpu.async_copy(src_hbm.at[pl.ds((b + 2) * BE, BE)],
                                 sb1, i1)
                pltpu.async_copy(dst_hbm.at[pl.ds((b + 2) * BE, BE)],
                                 db1, i1)
                pltpu.async_copy(ee_hbm.at[pl.ds((b + 2) * BE, BE)],
                                 eb1, i1)

            return carry

        lax.fori_loop(0, NBE // 2, pair, 0)
        pltpu.sync_copy(acc_v, out_hbm.at[pl.ds(w * CPT * NP_, CPT * NP_)])

    return _sc_deg, _sc_agg, _sc_edge_e, _sc_softmax_num, _sc_wagg


_BC = 512


def _tc1_body(degp_ref, ut_ref, y0t_ref, normt_ref, degc_ref):
    dp = degp_ref[...]
    deg = dp[0:1, :] + dp[1:2, :]
    degc = jnp.maximum(deg, 1.0)
    normt = lax.rsqrt(degc)
    degc_ref[...] = degc
    normt_ref[...] = normt
    y0t_ref[...] = ut_ref[...] * normt


def _tc1(deg_parts, u_t):
    return pl.pallas_call(
        _tc1_body,
        grid=(NP_ // _BC,),
        in_specs=[
            pl.BlockSpec((2, _BC), lambda i: (0, i)),
            pl.BlockSpec((D, _BC), lambda i: (0, i)),
        ],
        out_specs=[
            pl.BlockSpec((D, _BC), lambda i: (0, i)),
            pl.BlockSpec((1, _BC), lambda i: (0, i)),
            pl.BlockSpec((1, _BC), lambda i: (0, i)),
        ],
        out_shape=[
            jax.ShapeDtypeStruct((D, NP_), _f32),
            jax.ShapeDtypeStruct((1, NP_), _f32),
            jax.ShapeDtypeStruct((1, NP_), _f32),
        ],
    )(deg_parts, u_t)


def _tc2_body(h1t_ref, normt_ref, ut_ref, lam_ref, x1t_ref, y1t_ref):
    rn = 2.0 / lam_ref[0, 0]
    normt = normt_ref[...]
    h1 = h1t_ref[...] * normt
    x1 = -rn * h1 + ut_ref[...] * (rn - 1.0)
    x1t_ref[...] = x1
    y1t_ref[...] = x1 * normt


def _tc2(h1_t, normt, u_t, lam):
    return pl.pallas_call(
        _tc2_body,
        grid=(NP_ // _BC,),
        in_specs=[
            pl.BlockSpec((D, _BC), lambda i: (0, i)),
            pl.BlockSpec((1, _BC), lambda i: (0, i)),
            pl.BlockSpec((D, _BC), lambda i: (0, i)),
            pl.BlockSpec((1, 1), lambda i: (0, 0)),
        ],
        out_specs=[
            pl.BlockSpec((D, _BC), lambda i: (0, i)),
            pl.BlockSpec((D, _BC), lambda i: (0, i)),
        ],
        out_shape=[
            jax.ShapeDtypeStruct((D, NP_), _f32),
            jax.ShapeDtypeStruct((D, NP_), _f32),
        ],
    )(h1_t, normt, u_t, lam)


def _tc3_body(h2t_ref, normt_ref, x1t_ref, ut_ref, lam_ref,
              w0_ref, w1_ref, w2_ref, bc_ref, ws_ref, bs_ref, wd_ref, bd_ref,
              fst_ref, fdt_ref):
    rn = 2.0 / lam_ref[0, 0]
    h2 = h2t_ref[...] * normt_ref[...]
    x1 = x1t_ref[...]
    u = ut_ref[...]
    x2 = -2.0 * rn * h2 + x1 * (2.0 * rn - 1.0) - u
    h = (jnp.dot(w0_ref[...], u, preferred_element_type=_f32)
         + jnp.dot(w1_ref[...], x1, preferred_element_type=_f32)
         + jnp.dot(w2_ref[...], x2, preferred_element_type=_f32)
         + bc_ref[...])
    h = jnp.maximum(h, 0.0)
    fst_ref[...] = (jnp.dot(ws_ref[...], h, preferred_element_type=_f32)
                    + bs_ref[...])
    fdt_ref[...] = (jnp.dot(wd_ref[...], h, preferred_element_type=_f32)
                    + bd_ref[...])


def _tc3(h2_t, normt, x1_t, u_t, lam, w0t, w1t, w2t, bct, wst, bst, wdt, bdt):
    full = lambda i: (0, 0)
    return pl.pallas_call(
        _tc3_body,
        grid=(NP_ // _BC,),
        in_specs=[
            pl.BlockSpec((D, _BC), lambda i: (0, i)),
            pl.BlockSpec((1, _BC), lambda i: (0, i)),
            pl.BlockSpec((D, _BC), lambda i: (0, i)),
            pl.BlockSpec((D, _BC), lambda i: (0, i)),
            pl.BlockSpec((1, 1), full),
            pl.BlockSpec((D, D), full),
            pl.BlockSpec((D, D), full),
            pl.BlockSpec((D, D), full),
            pl.BlockSpec((D, 1), full),
            pl.BlockSpec((D, D), full),
            pl.BlockSpec((D, 1), full),
            pl.BlockSpec((D, D), full),
            pl.BlockSpec((D, 1), full),
        ],
        out_specs=[
            pl.BlockSpec((D, _BC), lambda i: (0, i)),
            pl.BlockSpec((D, _BC), lambda i: (0, i)),
        ],
        out_shape=[
            jax.ShapeDtypeStruct((D, NP_), _f32),
            jax.ShapeDtypeStruct((D, NP_), _f32),
        ],
    )(h2_t, normt, x1_t, u_t, lam, w0t, w1t, w2t, bct, wst, bst, wdt, bdt)


def _tc4_body(ot_ref, sp_ref, out_ref):
    sp = sp_ref[...]
    sden = sp[0:1, :] + sp[1:2, :]
    sden = jnp.where(sden > 0.0, sden, 1.0)
    out_ref[...] = ot_ref[...] / sden


def _tc4(out_t, s_parts):
    return pl.pallas_call(
        _tc4_body,
        grid=(NP_ // _BC,),
        in_specs=[
            pl.BlockSpec((D, _BC), lambda i: (0, i)),
            pl.BlockSpec((2, _BC), lambda i: (0, i)),
        ],
        out_specs=pl.BlockSpec((D, _BC), lambda i: (0, i)),
        out_shape=jax.ShapeDtypeStruct((D, NP_), _f32),
    )(out_t, s_parts)


def kernel(u, edge_index, lambda_max, W_cheb, b_cheb, W_src, b_src,
           W_dst, b_dst, attn):
    sc_deg, sc_agg, sc_edge_e, sc_softmax_num, sc_wagg = _sc_kernels()

    u_t = jnp.pad(u, ((0, NP_ - N), (0, 0))).T
    pad_e = EP - E
    src = jnp.concatenate([edge_index[0],
                           jnp.full((pad_e,), NP_ - 1, jnp.int32)])
    dst = jnp.concatenate([edge_index[1],
                           jnp.full((pad_e,), NP_ - 1, jnp.int32)])
    src2d = src.reshape(NW, NCH, CH)
    dst2d = dst.reshape(NW, NCH, CH)
    z1 = jnp.zeros((NP_,), _f32)
    lam = lambda_max.reshape(1, 1)
    w0t = W_cheb[0 * D:1 * D].T
    w1t = W_cheb[1 * D:2 * D].T
    w2t = W_cheb[2 * D:3 * D].T
    bct = b_cheb.reshape(D, 1)
    wst = W_src.T
    bst = b_src.reshape(D, 1)
    wdt = W_dst.T
    bdt = b_dst.reshape(D, 1)
    attn_v = attn.reshape(D)

    deg_parts = sc_deg(dst2d, z1)
    y0_t, normt, degc = _tc1(deg_parts, u_t)
    h1_t = sc_agg(y0_t.reshape(D * NP_), src, dst).reshape(D, NP_)
    x1_t, y1_t = _tc2(h1_t, normt, u_t, lam)
    h2_t = sc_agg(y1_t.reshape(D * NP_), src, dst).reshape(D, NP_)
    fs_t, fd_t = _tc3(h2_t, normt, x1_t, u_t, lam, w0t, w1t, w2t, bct,
                      wst, bst, wdt, bdt)
    fs = fs_t.T
    fd = fd_t.T

    e_edges, se_parts = sc_edge_e(fs, fd, src2d, dst2d, attn_v, z1)
    ee_edges, s_parts = sc_softmax_num(e_edges, dst2d, se_parts,
                                       degc.reshape(NP_), z1)
    out_t = sc_wagg(fs_t.reshape(D * NP_), src, dst,
                    ee_edges.reshape(EP)).reshape(D, NP_)
    out = _tc4(out_t, s_parts)
    return out.T[:N]


# speedup vs baseline: 1.2464x; 1.2464x over previous
import functools

import jax
import jax.numpy as jnp
from jax import lax
from jax.experimental import pallas as pl
from jax.experimental.pallas import tpu as pltpu
from jax.experimental.pallas import tpu_sc as plsc

N = 10000
E = 320000
D = 128
NP_ = 10240
NC, NS, L = 2, 16, 16
NW = NC * NS
CPT = D // NW
EPW = 10240
EP = EPW * NW
CH = 128
NCH = EPW // CH
BE = 4096
NBE = EP // BE
RPT = NP_ // NS

_f32 = jnp.float32
_params = pltpu.CompilerParams(use_tc_tiling_on_sc=False,
                               needs_layout_passes=False)


def _wid():
    c = lax.axis_index("c")
    s = lax.axis_index("s")
    return c, s, c * NS + s


@functools.cache
def _sc_kernels():
    mesh = plsc.VectorSubcoreMesh(core_axis_name="c", subcore_axis_name="s",
                                  num_cores=NC, num_subcores=NS)

    def _reduce_tiles(tab_v, red_v, stage_sh, out_ref, c, s):
        pltpu.sync_copy(tab_v, stage_sh.at[s])
        plsc.subcore_barrier()
        for t in range(NS):
            pltpu.sync_copy(stage_sh.at[t, pl.ds(s * RPT, RPT)], red_v)
            if t == 0:
                def cp0(k, carry):
                    tab_v[pl.ds(k * L, L)] = red_v[pl.ds(k * L, L)]
                    return carry
                lax.fori_loop(0, RPT // L, cp0, 0)
            else:
                def acc_t(k, carry):
                    tab_v[pl.ds(k * L, L)] = (tab_v[pl.ds(k * L, L)]
                                              + red_v[pl.ds(k * L, L)])
                    return carry
                lax.fori_loop(0, RPT // L, acc_t, 0)
        pltpu.sync_copy(tab_v.at[pl.ds(0, RPT)],
                        out_ref.at[c, pl.ds(s * RPT, RPT)])

    @functools.partial(
        pl.kernel,
        out_type=jax.ShapeDtypeStruct((NC, NP_), _f32),
        mesh=mesh,
        compiler_params=_params,
        scratch_types=[
            pltpu.VMEM((NCH, CH), jnp.int32),
            pltpu.VMEM((NP_,), _f32),
            pltpu.VMEM((RPT,), _f32),
            pltpu.VMEM_SHARED((NS, NP_), _f32),
        ],
    )
    def _sc_deg(dst_hbm, z1_hbm, deg_out, dst_v, tab_v, red_v, stage_sh):
        c, s, w = _wid()
        pltpu.sync_copy(dst_hbm.at[w], dst_v)
        pltpu.sync_copy(z1_hbm, tab_v)
        ones = jnp.ones((L,), _f32)

        def chunk(j, carry):
            for k in range(CH // L):
                dv = dst_v[j, pl.ds(k * L, L)]
                plsc.addupdate_scatter(tab_v, [dv], ones)
            return carry

        lax.fori_loop(0, NCH, chunk, 0)
        _reduce_tiles(tab_v, red_v, stage_sh, deg_out, c, s)

    DQ = 32
    NQ = D // DQ

    @functools.partial(
        pl.kernel,
        out_type=jax.ShapeDtypeStruct((NC, NQ, NP_, DQ), _f32),
        mesh=mesh,
        compiler_params=_params,
        scratch_types=[
            pltpu.VMEM((NCH, CH), jnp.int32),
            pltpu.VMEM((NCH, CH), jnp.int32),
            pltpu.VMEM((CH, DQ), _f32),
            pltpu.VMEM((CH, DQ), _f32),
            pltpu.VMEM_SHARED((NP_, DQ), _f32),
            pltpu.VMEM_SHARED((NP_, DQ), _f32),
            pltpu.SemaphoreType.DMA,
            pltpu.SemaphoreType.DMA,
        ],
    )
    def _sc_agg(yq_hbm, src_hbm, dst_hbm, zq_hbm, out_hbm,
                src_v, dst_v, buf0, buf1, tab_sh, acc_sh, g0, g1):
        c, s, w = _wid()
        pltpu.sync_copy(src_hbm.at[w], src_v)
        pltpu.sync_copy(dst_hbm.at[w], dst_v)
        for q in range(NQ):
            pltpu.sync_copy(yq_hbm.at[q, pl.ds(s * RPT, RPT)],
                            tab_sh.at[pl.ds(s * RPT, RPT)])
            pltpu.sync_copy(zq_hbm.at[pl.ds(s * RPT, RPT)],
                            acc_sh.at[pl.ds(s * RPT, RPT)])
            plsc.subcore_barrier()
            pltpu.async_copy(tab_sh.at[src_v.at[0]], buf0, g0)
            pltpu.async_copy(tab_sh.at[src_v.at[1]], buf1, g1)

            def pair(jj, carry):
                a = 2 * jj
                b = a + 1
                pltpu.make_async_copy(tab_sh.at[src_v.at[a]],
                                      buf0, g0).wait()
                pltpu.sync_copy(buf0, acc_sh.at[dst_v.at[a]], add=True)

                @pl.when(jj < NCH // 2 - 1)
                def _():
                    pltpu.async_copy(tab_sh.at[src_v.at[a + 2]], buf0, g0)

                pltpu.make_async_copy(tab_sh.at[src_v.at[b]],
                                      buf1, g1).wait()
                pltpu.sync_copy(buf1, acc_sh.at[dst_v.at[b]], add=True)

                @pl.when(jj < NCH // 2 - 1)
                def _():
                    pltpu.async_copy(tab_sh.at[src_v.at[b + 2]], buf1, g1)

                return carry

            lax.fori_loop(0, NCH // 2, pair, 0)
            plsc.subcore_barrier()
            pltpu.sync_copy(acc_sh.at[pl.ds(s * RPT, RPT)],
                            out_hbm.at[c, q, pl.ds(s * RPT, RPT)])

    @functools.partial(
        pl.kernel,
        out_type=(jax.ShapeDtypeStruct((NW, NCH, CH), _f32),
                  jax.ShapeDtypeStruct((NC, NP_), _f32)),
        mesh=mesh,
        compiler_params=_params,
        scratch_types=[
            pltpu.VMEM((NCH, CH), jnp.int32),
            pltpu.VMEM((NCH, CH), jnp.int32),
            pltpu.VMEM((CH, D), _f32),
            pltpu.VMEM((CH, D), _f32),
            pltpu.VMEM((CH, D), _f32),
            pltpu.VMEM((CH, D), _f32),
            pltpu.VMEM((NCH, CH), _f32),
            pltpu.VMEM((D,), _f32),
            pltpu.VMEM((NP_,), _f32),
            pltpu.VMEM((RPT,), _f32),
            pltpu.VMEM_SHARED((NS, NP_), _f32),
            pltpu.SemaphoreType.DMA,
            pltpu.SemaphoreType.DMA,
            pltpu.SemaphoreType.DMA,
            pltpu.SemaphoreType.DMA,
        ],
    )
    def _sc_edge_e(fs_hbm, fd_hbm, src_hbm, dst_hbm, attn_hbm, z1_hbm,
                   e_out, se_out,
                   src_v, dst_v, fsb0, fsb1, fdb0, fdb1, e_vm, attn_v,
                   tab_v, red_v, stage_sh,
                   gs0, gs1, gd0, gd1):
        c, s, w = _wid()
        pltpu.sync_copy(src_hbm.at[w], src_v)
        pltpu.sync_copy(dst_hbm.at[w], dst_v)
        pltpu.sync_copy(attn_hbm, attn_v)
        pltpu.sync_copy(z1_hbm, tab_v)
        pltpu.async_copy(fs_hbm.at[src_v.at[0]], fsb0, gs0)
        pltpu.async_copy(fd_hbm.at[dst_v.at[0]], fdb0, gd0)
        pltpu.async_copy(fs_hbm.at[src_v.at[1]], fsb1, gs1)
        pltpu.async_copy(fd_hbm.at[dst_v.at[1]], fdb1, gd1)

        lane = lax.iota(jnp.int32, L)

        def compute_chunk(j, fsb, fdb):
            def grp16(g, carry):
                accs = []
                for rr in range(L):
                    r = g * L + rr
                    acc = jnp.zeros((L,), _f32)
                    for dg in range(D // L):
                        x = (fsb[r, pl.ds(dg * L, L)]
                             + fdb[r, pl.ds(dg * L, L)])
                        t = jnp.where(x > 0.0, x, 0.2 * x)
                        acc = acc + t * attn_v[pl.ds(dg * L, L)]
                    accs.append(acc)
                vacc = jnp.zeros((L,), _f32)
                for rr in range(L):
                    vacc = jnp.where(lane == rr, jnp.sum(accs[rr]), vacc)
                e_vm[j, pl.ds(g * L, L)] = vacc
                dv = dst_v[j, pl.ds(g * L, L)]
                plsc.addupdate_scatter(tab_v, [dv], vacc)
                return carry

            lax.fori_loop(0, CH // L, grp16, 0)

        def pair(jj, carry):
            a = 2 * jj
            b = a + 1
            pltpu.make_async_copy(fs_hbm.at[src_v.at[a]], fsb0, gs0).wait()
            pltpu.make_async_copy(fd_hbm.at[dst_v.at[a]], fdb0, gd0).wait()
            compute_chunk(a, fsb0, fdb0)

            @pl.when(jj < NCH // 2 - 1)
            def _():
                pltpu.async_copy(fs_hbm.at[src_v.at[a + 2]], fsb0, gs0)
                pltpu.async_copy(fd_hbm.at[dst_v.at[a + 2]], fdb0, gd0)

            pltpu.make_async_copy(fs_hbm.at[src_v.at[b]], fsb1, gs1).wait()
            pltpu.make_async_copy(fd_hbm.at[dst_v.at[b]], fdb1, gd1).wait()
            compute_chunk(b, fsb1, fdb1)

            @pl.when(jj < NCH // 2 - 1)
            def _():
                pltpu.async_copy(fs_hbm.at[src_v.at[b + 2]], fsb1, gs1)
                pltpu.async_copy(fd_hbm.at[dst_v.at[b + 2]], fdb1, gd1)

            return carry

        lax.fori_loop(0, NCH // 2, pair, 0)
        pltpu.sync_copy(e_vm, e_out.at[w])
        _reduce_tiles(tab_v, red_v, stage_sh, se_out, c, s)

    @functools.partial(
        pl.kernel,
        out_type=(jax.ShapeDtypeStruct((NW, NCH, CH), _f32),
                  jax.ShapeDtypeStruct((NC, NP_), _f32)),
        mesh=mesh,
        compiler_params=_params,
        scratch_types=[
            pltpu.VMEM((NCH, CH), jnp.int32),
            pltpu.VMEM((NCH, CH), _f32),
            pltpu.VMEM((NCH, CH), _f32),
            pltpu.VMEM((NP_,), _f32),
            pltpu.VMEM((NP_,), _f32),
            pltpu.VMEM((RPT,), _f32),
            pltpu.VMEM_SHARED((NS, NP_), _f32),
        ],
    )
    def _sc_softmax_num(e_hbm, dst_hbm, se_hbm, degc_hbm, z1_hbm,
                        ee_out, s_out,
                        dst_v, e_vm, ee_vm, b_tab, tab_v, red_v, stage_sh):
        c, s, w = _wid()
        pltpu.sync_copy(dst_hbm.at[w], dst_v)
        pltpu.sync_copy(e_hbm.at[w], e_vm)
        pltpu.sync_copy(se_hbm.at[0], b_tab)
        pltpu.sync_copy(se_hbm.at[1], tab_v)

        def add_grp(k, carry):
            b_tab[pl.ds(k * L, L)] = (b_tab[pl.ds(k * L, L)]
                                      + tab_v[pl.ds(k * L, L)])
            return carry

        lax.fori_loop(0, NP_ // L, add_grp, 0)
        pltpu.sync_copy(degc_hbm, tab_v)

        def div_grp(k, carry):
            b_tab[pl.ds(k * L, L)] = (b_tab[pl.ds(k * L, L)]
                                      / tab_v[pl.ds(k * L, L)])
            return carry

        lax.fori_loop(0, NP_ // L, div_grp, 0)
        pltpu.sync_copy(z1_hbm, tab_v)

        def chunk(j, carry):
            def grp(k, carry2):
                dv = dst_v[j, pl.ds(k * L, L)]
                bv = plsc.load_gather(b_tab, [dv])
                ee = jnp.exp(e_vm[j, pl.ds(k * L, L)] - bv)
                ee_vm[j, pl.ds(k * L, L)] = ee
                plsc.addupdate_scatter(tab_v, [dv], ee)
                return carry2

            lax.fori_loop(0, CH // L, grp, 0)
            return carry

        lax.fori_loop(0, NCH, chunk, 0)
        pltpu.sync_copy(ee_vm, ee_out.at[w])
        _reduce_tiles(tab_v, red_v, stage_sh, s_out, c, s)

    @functools.partial(
        pl.kernel,
        out_type=jax.ShapeDtypeStruct((D * NP_,), _f32),
        mesh=mesh,
        compiler_params=_params,
        scratch_types=[
            pltpu.VMEM((CPT * NP_,), _f32),
            pltpu.VMEM((CPT * NP_,), _f32),
            pltpu.VMEM((BE,), jnp.int32),
            pltpu.VMEM((BE,), jnp.int32),
            pltpu.VMEM((BE,), _f32),
            pltpu.VMEM((BE,), jnp.int32),
            pltpu.VMEM((BE,), jnp.int32),
            pltpu.VMEM((BE,), _f32),
            pltpu.SemaphoreType.DMA,
            pltpu.SemaphoreType.DMA,
        ],
    )
    def _sc_wagg(fst_hbm, src_hbm, dst_hbm, ee_hbm, out_hbm,
                 tab_v, acc_v, sb0, db0, eb0, sb1, db1, eb1, i0, i1):
        c, s, w = _wid()
        pltpu.sync_copy(fst_hbm.at[pl.ds(w * CPT * NP_, CPT * NP_)], tab_v)

        def zero(k, carry):
            acc_v[pl.ds(k * L, L)] = jnp.zeros((L,), _f32)
            return carry

        lax.fori_loop(0, CPT * NP_ // L, zero, 0)
        off = [jnp.full((L,), cc * NP_, jnp.int32) for cc in range(CPT)]
        UN = 4

        def process(sb, db, eb):
            def grp(g, carry):
                for uu in range(UN):
                    srcv = sb[pl.ds((g * UN + uu) * L, L)]
                    dstv = db[pl.ds((g * UN + uu) * L, L)]
                    eev = eb[pl.ds((g * UN + uu) * L, L)]
                    for cc in range(CPT):
                        v = plsc.load_gather(tab_v, [srcv + off[cc]]) * eev
                        plsc.addupdate_scatter(acc_v, [dstv + off[cc]], v)
                return carry

            lax.fori_loop(0, BE // L // UN, grp, 0)

        pltpu.async_copy(src_hbm.at[pl.ds(0, BE)], sb0, i0)
        pltpu.async_copy(dst_hbm.at[pl.ds(0, BE)], db0, i0)
        pltpu.async_copy(ee_hbm.at[pl.ds(0, BE)], eb0, i0)
        pltpu.async_copy(src_hbm.at[pl.ds(BE, BE)], sb1, i1)
        pltpu.async_copy(dst_hbm.at[pl.ds(BE, BE)], db1, i1)
        pltpu.async_copy(ee_hbm.at[pl.ds(BE, BE)], eb1, i1)

        def pair(i, carry):
            a = 2 * i
            b = a + 1
            pltpu.make_async_copy(src_hbm.at[pl.ds(0, BE)], sb0, i0).wait()
            pltpu.make_async_copy(dst_hbm.at[pl.ds(0, BE)], db0, i0).wait()
            pltpu.make_async_copy(ee_hbm.at[pl.ds(0, BE)], eb0, i0).wait()
            process(sb0, db0, eb0)

            @pl.when(i < NBE // 2 - 1)
            def _():
                pltpu.async_copy(src_hbm.at[pl.ds((a + 2) * BE, BE)],
                                 sb0, i0)
                pltpu.async_copy(dst_hbm.at[pl.ds((a + 2) * BE, BE)],
                                 db0, i0)
                pltpu.async_copy(ee_hbm.at[pl.ds((a + 2) * BE, BE)],
                                 eb0, i0)

            pltpu.make_async_copy(src_hbm.at[pl.ds(0, BE)], sb1, i1).wait()
            pltpu.make_async_copy(dst_hbm.at[pl.ds(0, BE)], db1, i1).wait()
            pltpu.make_async_copy(ee_hbm.at[pl.ds(0, BE)], eb1, i1).wait()
            process(sb1, db1, eb1)

            @pl.when(i < NBE // 2 - 1)
            def _():
                pltpu.async_copy(src_hbm.at[pl.ds((b + 2) * BE, BE)],
                                 sb1, i1)
                pltpu.async_copy(dst_hbm.at[pl.ds((b + 2) * BE, BE)],
                                 db1, i1)
                pltpu.async_copy(ee_hbm.at[pl.ds((b + 2) * BE, BE)],
                                 eb1, i1)

            return carry

        lax.fori_loop(0, NBE // 2, pair, 0)
        pltpu.sync_copy(acc_v, out_hbm.at[pl.ds(w * CPT * NP_, CPT * NP_)])

    return _sc_deg, _sc_agg, _sc_edge_e, _sc_softmax_num, _sc_wagg


_BR = 512
_BC = 512
_NQ = 4
_DQ = 32


def _tc1_body(degp_ref, u_ref, y0_ref, norm_ref, degc_ref):
    deg = degp_ref[0] + degp_ref[1]
    degc = jnp.maximum(deg, 1.0)
    norm = lax.rsqrt(degc)
    degc_ref[...] = degc
    norm_ref[...] = norm
    y0_ref[...] = u_ref[...] * norm


def _tc1(deg_parts, u_pad):
    return pl.pallas_call(
        _tc1_body,
        grid=(NP_ // _BR,),
        in_specs=[
            pl.BlockSpec((2, _BR, 1), lambda i: (0, i, 0)),
            pl.BlockSpec((_BR, D), lambda i: (i, 0)),
        ],
        out_specs=[
            pl.BlockSpec((_BR, D), lambda i: (i, 0)),
            pl.BlockSpec((_BR, 1), lambda i: (i, 0)),
            pl.BlockSpec((_BR, 1), lambda i: (i, 0)),
        ],
        out_shape=[
            jax.ShapeDtypeStruct((NP_, D), _f32),
            jax.ShapeDtypeStruct((NP_, 1), _f32),
            jax.ShapeDtypeStruct((NP_, 1), _f32),
        ],
    )(deg_parts, u_pad)


def _tc2_body(h1p_ref, norm_ref, u_ref, lam_ref, x1_ref, y1_ref):
    rn = 2.0 / lam_ref[0, 0]
    h1 = (h1p_ref[0] + h1p_ref[1]) * norm_ref[...]
    x1 = -rn * h1 + u_ref[...] * (rn - 1.0)
    x1_ref[...] = x1
    y1_ref[...] = x1 * norm_ref[...]


def _tc2(h1_parts, norm, u_pad, lam):
    return pl.pallas_call(
        _tc2_body,
        grid=(NP_ // _BR,),
        in_specs=[
            pl.BlockSpec((2, _BR, D), lambda i: (0, i, 0)),
            pl.BlockSpec((_BR, 1), lambda i: (i, 0)),
            pl.BlockSpec((_BR, D), lambda i: (i, 0)),
            pl.BlockSpec((1, 1), lambda i: (0, 0)),
        ],
        out_specs=[
            pl.BlockSpec((_BR, D), lambda i: (i, 0)),
            pl.BlockSpec((_BR, D), lambda i: (i, 0)),
        ],
        out_shape=[
            jax.ShapeDtypeStruct((NP_, D), _f32),
            jax.ShapeDtypeStruct((NP_, D), _f32),
        ],
    )(h1_parts, norm, u_pad, lam)


def _tc3_body(h2p_ref, norm_ref, x1_ref, u_ref, lam_ref,
              w0_ref, w1_ref, w2_ref, bc_ref, ws_ref, bs_ref, wd_ref, bd_ref,
              fs_ref, fd_ref):
    rn = 2.0 / lam_ref[0, 0]
    h2 = (h2p_ref[0] + h2p_ref[1]) * norm_ref[...]
    x1 = x1_ref[...]
    u = u_ref[...]
    x2 = -2.0 * rn * h2 + x1 * (2.0 * rn - 1.0) - u
    h = (jnp.dot(u, w0_ref[...], preferred_element_type=_f32)
         + jnp.dot(x1, w1_ref[...], preferred_element_type=_f32)
         + jnp.dot(x2, w2_ref[...], preferred_element_type=_f32)
         + bc_ref[...])
    h = jnp.maximum(h, 0.0)
    fs_ref[...] = (jnp.dot(h, ws_ref[...], preferred_element_type=_f32)
                   + bs_ref[...])
    fd_ref[...] = (jnp.dot(h, wd_ref[...], preferred_element_type=_f32)
                   + bd_ref[...])


def _tc3(h2_parts, norm, x1, u_pad, lam, w0, w1, w2, bc, ws, bs, wd, bd):
    full = lambda i: (0, 0)
    return pl.pallas_call(
        _tc3_body,
        grid=(NP_ // _BR,),
        in_specs=[
            pl.BlockSpec((2, _BR, D), lambda i: (0, i, 0)),
            pl.BlockSpec((_BR, 1), lambda i: (i, 0)),
            pl.BlockSpec((_BR, D), lambda i: (i, 0)),
            pl.BlockSpec((_BR, D), lambda i: (i, 0)),
            pl.BlockSpec((1, 1), full),
            pl.BlockSpec((D, D), full),
            pl.BlockSpec((D, D), full),
            pl.BlockSpec((D, D), full),
            pl.BlockSpec((1, D), full),
            pl.BlockSpec((D, D), full),
            pl.BlockSpec((1, D), full),
            pl.BlockSpec((D, D), full),
            pl.BlockSpec((1, D), full),
        ],
        out_specs=[
            pl.BlockSpec((_BR, D), lambda i: (i, 0)),
            pl.BlockSpec((_BR, D), lambda i: (i, 0)),
        ],
        out_shape=[
            jax.ShapeDtypeStruct((NP_, D), _f32),
            jax.ShapeDtypeStruct((NP_, D), _f32),
        ],
    )(h2_parts, norm, x1, u_pad, lam, w0, w1, w2, bc, ws, bs, wd, bd)


def _tc4_body(ot_ref, sp_ref, out_ref):
    sp = sp_ref[...]
    sden = sp[0:1, :] + sp[1:2, :]
    sden = jnp.where(sden > 0.0, sden, 1.0)
    out_ref[...] = ot_ref[...] / sden


def _tc4(out_t, s_parts):
    return pl.pallas_call(
        _tc4_body,
        grid=(NP_ // _BC,),
        in_specs=[
            pl.BlockSpec((D, _BC), lambda i: (0, i)),
            pl.BlockSpec((2, _BC), lambda i: (0, i)),
        ],
        out_specs=pl.BlockSpec((D, _BC), lambda i: (0, i)),
        out_shape=jax.ShapeDtypeStruct((D, NP_), _f32),
    )(out_t, s_parts)


def kernel(u, edge_index, lambda_max, W_cheb, b_cheb, W_src, b_src,
           W_dst, b_dst, attn):
    sc_deg, sc_agg, sc_edge_e, sc_softmax_num, sc_wagg = _sc_kernels()

    u_pad = jnp.pad(u, ((0, NP_ - N), (0, 0)))
    pad_e = EP - E
    src = jnp.concatenate([edge_index[0],
                           jnp.full((pad_e,), NP_ - 1, jnp.int32)])
    dst = jnp.concatenate([edge_index[1],
                           jnp.full((pad_e,), NP_ - 1, jnp.int32)])
    src2d = src.reshape(NW, NCH, CH)
    dst2d = dst.reshape(NW, NCH, CH)
    z1 = jnp.zeros((NP_,), _f32)
    zq = jnp.zeros((NP_, _DQ), _f32)
    lam = lambda_max.reshape(1, 1)
    w0 = W_cheb[0 * D:1 * D]
    w1 = W_cheb[1 * D:2 * D]
    w2 = W_cheb[2 * D:3 * D]
    bc = b_cheb.reshape(1, D)
    bs = b_src.reshape(1, D)
    bd = b_dst.reshape(1, D)
    attn_v = attn.reshape(D)

    def _quarters(x):
        return x.reshape(NP_, _NQ, _DQ).transpose(1, 0, 2)

    def _merge(parts):
        return jnp.concatenate([parts[:, q] for q in range(_NQ)], axis=-1)

    deg_parts = sc_deg(dst2d, z1)
    y0, norm, degc = _tc1(deg_parts.reshape(2, NP_, 1), u_pad)
    h1_parts = sc_agg(_quarters(y0), src2d, dst2d, zq)
    x1, y1 = _tc2(_merge(h1_parts), norm, u_pad, lam)
    h2_parts = sc_agg(_quarters(y1), src2d, dst2d, zq)
    fs, fd = _tc3(_merge(h2_parts), norm, x1, u_pad, lam, w0, w1, w2, bc,
                  W_src, bs, W_dst, bd)

    e_edges, se_parts = sc_edge_e(fs, fd, src2d, dst2d, attn_v, z1)
    ee_edges, s_parts = sc_softmax_num(e_edges, dst2d, se_parts,
                                       degc.reshape(NP_), z1)
    out_t = sc_wagg(fs.T.reshape(D * NP_), src, dst,
                    ee_edges.reshape(EP)).reshape(D, NP_)
    out = _tc4(out_t, s_parts)
    return out.T[:N]


# speedup vs baseline: 1.5451x; 1.2397x over previous
import functools

import jax
import jax.numpy as jnp
from jax import lax
from jax.experimental import pallas as pl
from jax.experimental.pallas import tpu as pltpu
from jax.experimental.pallas import tpu_sc as plsc

N = 10000
E = 320000
D = 128
NP_ = 10240
NC, NS, L = 2, 16, 16
NW = NC * NS
CPT = D // NW
EPW = 10240
EP = EPW * NW
CH = 128
NCH = EPW // CH
BE = 4096
NBE = EP // BE
RPT = NP_ // NS

_f32 = jnp.float32
_params = pltpu.CompilerParams(use_tc_tiling_on_sc=False,
                               needs_layout_passes=False)


def _wid():
    c = lax.axis_index("c")
    s = lax.axis_index("s")
    return c, s, c * NS + s


@functools.cache
def _sc_kernels():
    mesh = plsc.VectorSubcoreMesh(core_axis_name="c", subcore_axis_name="s",
                                  num_cores=NC, num_subcores=NS)

    def _reduce_tiles(tab_v, red_v, stage_sh, out_ref, c, s):
        pltpu.sync_copy(tab_v, stage_sh.at[s])
        plsc.subcore_barrier()
        for t in range(NS):
            pltpu.sync_copy(stage_sh.at[t, pl.ds(s * RPT, RPT)], red_v)
            if t == 0:
                def cp0(k, carry):
                    tab_v[pl.ds(k * L, L)] = red_v[pl.ds(k * L, L)]
                    return carry
                lax.fori_loop(0, RPT // L, cp0, 0)
            else:
                def acc_t(k, carry):
                    tab_v[pl.ds(k * L, L)] = (tab_v[pl.ds(k * L, L)]
                                              + red_v[pl.ds(k * L, L)])
                    return carry
                lax.fori_loop(0, RPT // L, acc_t, 0)
        pltpu.sync_copy(tab_v.at[pl.ds(0, RPT)],
                        out_ref.at[c, pl.ds(s * RPT, RPT)])

    @functools.partial(
        pl.kernel,
        out_type=jax.ShapeDtypeStruct((NC, NP_), _f32),
        mesh=mesh,
        compiler_params=_params,
        scratch_types=[
            pltpu.VMEM((NCH, CH), jnp.int32),
            pltpu.VMEM((NP_,), _f32),
            pltpu.VMEM((RPT,), _f32),
            pltpu.VMEM_SHARED((NS, NP_), _f32),
        ],
    )
    def _sc_deg(dst_hbm, z1_hbm, deg_out, dst_v, tab_v, red_v, stage_sh):
        c, s, w = _wid()
        pltpu.sync_copy(dst_hbm.at[w], dst_v)
        pltpu.sync_copy(z1_hbm, tab_v)
        ones = jnp.ones((L,), _f32)

        def chunk(j, carry):
            for k in range(CH // L):
                dv = dst_v[j, pl.ds(k * L, L)]
                plsc.addupdate_scatter(tab_v, [dv], ones)
            return carry

        lax.fori_loop(0, NCH, chunk, 0)
        _reduce_tiles(tab_v, red_v, stage_sh, deg_out, c, s)

    DQ = 32
    NQ = D // DQ

    @functools.partial(
        pl.kernel,
        out_type=jax.ShapeDtypeStruct((NC, NQ, NP_, DQ), _f32),
        mesh=mesh,
        compiler_params=_params,
        scratch_types=[
            pltpu.VMEM((NCH, CH), jnp.int32),
            pltpu.VMEM((NCH, CH), jnp.int32),
            pltpu.VMEM((CH, DQ), _f32),
            pltpu.VMEM((CH, DQ), _f32),
            pltpu.VMEM_SHARED((NP_, DQ), _f32),
            pltpu.VMEM_SHARED((NP_, DQ), _f32),
            pltpu.SemaphoreType.DMA,
            pltpu.SemaphoreType.DMA,
        ],
    )
    def _sc_agg(yq_hbm, src_hbm, dst_hbm, zq_hbm, out_hbm,
                src_v, dst_v, buf0, buf1, tab_sh, acc_sh, g0, g1):
        c, s, w = _wid()
        pltpu.sync_copy(src_hbm.at[w], src_v)
        pltpu.sync_copy(dst_hbm.at[w], dst_v)
        for q in range(NQ):
            pltpu.sync_copy(yq_hbm.at[q, pl.ds(s * RPT, RPT)],
                            tab_sh.at[pl.ds(s * RPT, RPT)])
            pltpu.sync_copy(zq_hbm.at[pl.ds(s * RPT, RPT)],
                            acc_sh.at[pl.ds(s * RPT, RPT)])
            plsc.subcore_barrier()
            pltpu.async_copy(tab_sh.at[src_v.at[0]], buf0, g0)
            pltpu.async_copy(tab_sh.at[src_v.at[1]], buf1, g1)

            def pair(jj, carry):
                a = 2 * jj
                b = a + 1
                pltpu.make_async_copy(tab_sh.at[src_v.at[a]],
                                      buf0, g0).wait()
                pltpu.sync_copy(buf0, acc_sh.at[dst_v.at[a]], add=True)

                @pl.when(jj < NCH // 2 - 1)
                def _():
                    pltpu.async_copy(tab_sh.at[src_v.at[a + 2]], buf0, g0)

                pltpu.make_async_copy(tab_sh.at[src_v.at[b]],
                                      buf1, g1).wait()
                pltpu.sync_copy(buf1, acc_sh.at[dst_v.at[b]], add=True)

                @pl.when(jj < NCH // 2 - 1)
                def _():
                    pltpu.async_copy(tab_sh.at[src_v.at[b + 2]], buf1, g1)

                return carry

            lax.fori_loop(0, NCH // 2, pair, 0)
            plsc.subcore_barrier()
            pltpu.sync_copy(acc_sh.at[pl.ds(s * RPT, RPT)],
                            out_hbm.at[c, q, pl.ds(s * RPT, RPT)])

    @functools.partial(
        pl.kernel,
        out_type=(jax.ShapeDtypeStruct((NW, NCH, CH), _f32),
                  jax.ShapeDtypeStruct((NC, NP_), _f32)),
        mesh=mesh,
        compiler_params=_params,
        scratch_types=[
            pltpu.VMEM((NCH, CH), jnp.int32),
            pltpu.VMEM((NCH, CH), jnp.int32),
            pltpu.VMEM((CH, D), _f32),
            pltpu.VMEM((CH, D), _f32),
            pltpu.VMEM((CH, D), _f32),
            pltpu.VMEM((CH, D), _f32),
            pltpu.VMEM((NCH, CH), _f32),
            pltpu.VMEM((D,), _f32),
            pltpu.VMEM((NP_,), _f32),
            pltpu.VMEM((RPT,), _f32),
            pltpu.VMEM_SHARED((NS, NP_), _f32),
            pltpu.SemaphoreType.DMA,
            pltpu.SemaphoreType.DMA,
            pltpu.SemaphoreType.DMA,
            pltpu.SemaphoreType.DMA,
        ],
    )
    def _sc_edge_e(fs_hbm, fd_hbm, src_hbm, dst_hbm, attn_hbm, z1_hbm,
                   e_out, se_out,
                   src_v, dst_v, fsb0, fsb1, fdb0, fdb1, e_vm, attn_v,
                   tab_v, red_v, stage_sh,
                   gs0, gs1, gd0, gd1):
        c, s, w = _wid()
        pltpu.sync_copy(src_hbm.at[w], src_v)
        pltpu.sync_copy(dst_hbm.at[w], dst_v)
        pltpu.sync_copy(attn_hbm, attn_v)
        pltpu.sync_copy(z1_hbm, tab_v)
        pltpu.async_copy(fs_hbm.at[src_v.at[0]], fsb0, gs0)
        pltpu.async_copy(fd_hbm.at[dst_v.at[0]], fdb0, gd0)
        pltpu.async_copy(fs_hbm.at[src_v.at[1]], fsb1, gs1)
        pltpu.async_copy(fd_hbm.at[dst_v.at[1]], fdb1, gd1)

        lane = lax.iota(jnp.int32, L)

        def compute_chunk(j, fsb, fdb):
            def grp16(g, carry):
                accs = []
                for rr in range(L):
                    r = g * L + rr
                    acc = jnp.zeros((L,), _f32)
                    for dg in range(D // L):
                        x = (fsb[r, pl.ds(dg * L, L)]
                             + fdb[r, pl.ds(dg * L, L)])
                        t = jnp.where(x > 0.0, x, 0.2 * x)
                        acc = acc + t * attn_v[pl.ds(dg * L, L)]
                    accs.append(acc)
                vacc = jnp.zeros((L,), _f32)
                for rr in range(L):
                    vacc = jnp.where(lane == rr, jnp.sum(accs[rr]), vacc)
                e_vm[j, pl.ds(g * L, L)] = vacc
                dv = dst_v[j, pl.ds(g * L, L)]
                plsc.addupdate_scatter(tab_v, [dv], vacc)
                return carry

            lax.fori_loop(0, CH // L, grp16, 0)

        def pair(jj, carry):
            a = 2 * jj
            b = a + 1
            pltpu.make_async_copy(fs_hbm.at[src_v.at[a]], fsb0, gs0).wait()
            pltpu.make_async_copy(fd_hbm.at[dst_v.at[a]], fdb0, gd0).wait()
            compute_chunk(a, fsb0, fdb0)

            @pl.when(jj < NCH // 2 - 1)
            def _():
                pltpu.async_copy(fs_hbm.at[src_v.at[a + 2]], fsb0, gs0)
                pltpu.async_copy(fd_hbm.at[dst_v.at[a + 2]], fdb0, gd0)

            pltpu.make_async_copy(fs_hbm.at[src_v.at[b]], fsb1, gs1).wait()
            pltpu.make_async_copy(fd_hbm.at[dst_v.at[b]], fdb1, gd1).wait()
            compute_chunk(b, fsb1, fdb1)

            @pl.when(jj < NCH // 2 - 1)
            def _():
                pltpu.async_copy(fs_hbm.at[src_v.at[b + 2]], fsb1, gs1)
                pltpu.async_copy(fd_hbm.at[dst_v.at[b + 2]], fdb1, gd1)

            return carry

        lax.fori_loop(0, NCH // 2, pair, 0)
        pltpu.sync_copy(e_vm, e_out.at[w])
        _reduce_tiles(tab_v, red_v, stage_sh, se_out, c, s)

    @functools.partial(
        pl.kernel,
        out_type=(jax.ShapeDtypeStruct((NW, NCH, CH), _f32),
                  jax.ShapeDtypeStruct((NC, NP_), _f32)),
        mesh=mesh,
        compiler_params=_params,
        scratch_types=[
            pltpu.VMEM((NCH, CH), jnp.int32),
            pltpu.VMEM((NCH, CH), _f32),
            pltpu.VMEM((NCH, CH), _f32),
            pltpu.VMEM((NP_,), _f32),
            pltpu.VMEM((NP_,), _f32),
            pltpu.VMEM((RPT,), _f32),
            pltpu.VMEM_SHARED((NS, NP_), _f32),
        ],
    )
    def _sc_softmax_num(e_hbm, dst_hbm, se_hbm, degc_hbm, z1_hbm,
                        ee_out, s_out,
                        dst_v, e_vm, ee_vm, b_tab, tab_v, red_v, stage_sh):
        c, s, w = _wid()
        pltpu.sync_copy(dst_hbm.at[w], dst_v)
        pltpu.sync_copy(e_hbm.at[w], e_vm)
        pltpu.sync_copy(se_hbm.at[0], b_tab)
        pltpu.sync_copy(se_hbm.at[1], tab_v)

        def add_grp(k, carry):
            b_tab[pl.ds(k * L, L)] = (b_tab[pl.ds(k * L, L)]
                                      + tab_v[pl.ds(k * L, L)])
            return carry

        lax.fori_loop(0, NP_ // L, add_grp, 0)
        pltpu.sync_copy(degc_hbm, tab_v)

        def div_grp(k, carry):
            b_tab[pl.ds(k * L, L)] = (b_tab[pl.ds(k * L, L)]
                                      / tab_v[pl.ds(k * L, L)])
            return carry

        lax.fori_loop(0, NP_ // L, div_grp, 0)
        pltpu.sync_copy(z1_hbm, tab_v)

        def chunk(j, carry):
            def grp(k, carry2):
                dv = dst_v[j, pl.ds(k * L, L)]
                bv = plsc.load_gather(b_tab, [dv])
                ee = jnp.exp(e_vm[j, pl.ds(k * L, L)] - bv)
                ee_vm[j, pl.ds(k * L, L)] = ee
                plsc.addupdate_scatter(tab_v, [dv], ee)
                return carry2

            lax.fori_loop(0, CH // L, grp, 0)
            return carry

        lax.fori_loop(0, NCH, chunk, 0)
        pltpu.sync_copy(ee_vm, ee_out.at[w])
        _reduce_tiles(tab_v, red_v, stage_sh, s_out, c, s)

    _dn = lax.GatherDimensionNumbers(offset_dims=(),
                                     collapsed_slice_dims=(0,),
                                     start_index_map=(0,))

    @functools.partial(
        pl.kernel,
        out_type=jax.ShapeDtypeStruct((NC, NQ, NP_, DQ), _f32),
        mesh=mesh,
        compiler_params=_params,
        scratch_types=[
            pltpu.VMEM((NCH, CH), jnp.int32),
            pltpu.VMEM((NCH, CH), jnp.int32),
            pltpu.VMEM((NCH, CH), _f32),
            pltpu.VMEM((CH, DQ), _f32),
            pltpu.VMEM((CH, DQ), _f32),
            pltpu.VMEM_SHARED((NP_, DQ), _f32),
            pltpu.VMEM_SHARED((NP_, DQ), _f32),
            pltpu.SemaphoreType.DMA,
            pltpu.SemaphoreType.DMA,
        ],
    )
    def _sc_wagg(fsq_hbm, src_hbm, dst_hbm, ee_hbm, zq_hbm, out_hbm,
                 src_v, dst_v, ee_vm, buf0, buf1, tab_sh, acc_sh, g0, g1):
        c, s, w = _wid()
        pltpu.sync_copy(src_hbm.at[w], src_v)
        pltpu.sync_copy(dst_hbm.at[w], dst_v)
        pltpu.sync_copy(ee_hbm.at[w], ee_vm)

        def scale_chunk(j, buf):
            def grp(g, carry):
                eev = ee_vm[j, pl.ds(g * L, L)]
                for rr in range(L):
                    r = g * L + rr
                    av = lax.gather(
                        eev, jnp.full((L, 1), rr, jnp.int32), _dn, (1,),
                        mode=lax.GatherScatterMode.PROMISE_IN_BOUNDS)
                    for dg in range(DQ // L):
                        buf[r, pl.ds(dg * L, L)] = (
                            buf[r, pl.ds(dg * L, L)] * av)
                return carry

            lax.fori_loop(0, CH // L, grp, 0)

        for q in range(NQ):
            pltpu.sync_copy(fsq_hbm.at[q, pl.ds(s * RPT, RPT)],
                            tab_sh.at[pl.ds(s * RPT, RPT)])
            pltpu.sync_copy(zq_hbm.at[pl.ds(s * RPT, RPT)],
                            acc_sh.at[pl.ds(s * RPT, RPT)])
            plsc.subcore_barrier()
            pltpu.async_copy(tab_sh.at[src_v.at[0]], buf0, g0)
            pltpu.async_copy(tab_sh.at[src_v.at[1]], buf1, g1)

            def pair(jj, carry):
                a = 2 * jj
                b = a + 1
                pltpu.make_async_copy(tab_sh.at[src_v.at[a]],
                                      buf0, g0).wait()
                scale_chunk(a, buf0)
                pltpu.sync_copy(buf0, acc_sh.at[dst_v.at[a]], add=True)

                @pl.when(jj < NCH // 2 - 1)
                def _():
                    pltpu.async_copy(tab_sh.at[src_v.at[a + 2]], buf0, g0)

                pltpu.make_async_copy(tab_sh.at[src_v.at[b]],
                                      buf1, g1).wait()
                scale_chunk(b, buf1)
                pltpu.sync_copy(buf1, acc_sh.at[dst_v.at[b]], add=True)

                @pl.when(jj < NCH // 2 - 1)
                def _():
                    pltpu.async_copy(tab_sh.at[src_v.at[b + 2]], buf1, g1)

                return carry

            lax.fori_loop(0, NCH // 2, pair, 0)
            plsc.subcore_barrier()
            pltpu.sync_copy(acc_sh.at[pl.ds(s * RPT, RPT)],
                            out_hbm.at[c, q, pl.ds(s * RPT, RPT)])

    return _sc_deg, _sc_agg, _sc_edge_e, _sc_softmax_num, _sc_wagg


_BR = 512
_BC = 512
_NQ = 4
_DQ = 32


def _tc1_body(degp_ref, u_ref, y0_ref, norm_ref, degc_ref):
    deg = degp_ref[0] + degp_ref[1]
    degc = jnp.maximum(deg, 1.0)
    norm = lax.rsqrt(degc)
    degc_ref[...] = degc
    norm_ref[...] = norm
    y0_ref[...] = u_ref[...] * norm


def _tc1(deg_parts, u_pad):
    return pl.pallas_call(
        _tc1_body,
        grid=(NP_ // _BR,),
        in_specs=[
            pl.BlockSpec((2, _BR, 1), lambda i: (0, i, 0)),
            pl.BlockSpec((_BR, D), lambda i: (i, 0)),
        ],
        out_specs=[
            pl.BlockSpec((_BR, D), lambda i: (i, 0)),
            pl.BlockSpec((_BR, 1), lambda i: (i, 0)),
            pl.BlockSpec((_BR, 1), lambda i: (i, 0)),
        ],
        out_shape=[
            jax.ShapeDtypeStruct((NP_, D), _f32),
            jax.ShapeDtypeStruct((NP_, 1), _f32),
            jax.ShapeDtypeStruct((NP_, 1), _f32),
        ],
    )(deg_parts, u_pad)


def _tc2_body(h1p_ref, norm_ref, u_ref, lam_ref, x1_ref, y1_ref):
    rn = 2.0 / lam_ref[0, 0]
    h1 = (h1p_ref[0] + h1p_ref[1]) * norm_ref[...]
    x1 = -rn * h1 + u_ref[...] * (rn - 1.0)
    x1_ref[...] = x1
    y1_ref[...] = x1 * norm_ref[...]


def _tc2(h1_parts, norm, u_pad, lam):
    return pl.pallas_call(
        _tc2_body,
        grid=(NP_ // _BR,),
        in_specs=[
            pl.BlockSpec((2, _BR, D), lambda i: (0, i, 0)),
            pl.BlockSpec((_BR, 1), lambda i: (i, 0)),
            pl.BlockSpec((_BR, D), lambda i: (i, 0)),
            pl.BlockSpec((1, 1), lambda i: (0, 0)),
        ],
        out_specs=[
            pl.BlockSpec((_BR, D), lambda i: (i, 0)),
            pl.BlockSpec((_BR, D), lambda i: (i, 0)),
        ],
        out_shape=[
            jax.ShapeDtypeStruct((NP_, D), _f32),
            jax.ShapeDtypeStruct((NP_, D), _f32),
        ],
    )(h1_parts, norm, u_pad, lam)


def _tc3_body(h2p_ref, norm_ref, x1_ref, u_ref, lam_ref,
              w0_ref, w1_ref, w2_ref, bc_ref, ws_ref, bs_ref, wd_ref, bd_ref,
              fs_ref, fd_ref):
    rn = 2.0 / lam_ref[0, 0]
    h2 = (h2p_ref[0] + h2p_ref[1]) * norm_ref[...]
    x1 = x1_ref[...]
    u = u_ref[...]
    x2 = -2.0 * rn * h2 + x1 * (2.0 * rn - 1.0) - u
    h = (jnp.dot(u, w0_ref[...], preferred_element_type=_f32)
         + jnp.dot(x1, w1_ref[...], preferred_element_type=_f32)
         + jnp.dot(x2, w2_ref[...], preferred_element_type=_f32)
         + bc_ref[...])
    h = jnp.maximum(h, 0.0)
    fs_ref[...] = (jnp.dot(h, ws_ref[...], preferred_element_type=_f32)
                   + bs_ref[...])
    fd_ref[...] = (jnp.dot(h, wd_ref[...], preferred_element_type=_f32)
                   + bd_ref[...])


def _tc3(h2_parts, norm, x1, u_pad, lam, w0, w1, w2, bc, ws, bs, wd, bd):
    full = lambda i: (0, 0)
    return pl.pallas_call(
        _tc3_body,
        grid=(NP_ // _BR,),
        in_specs=[
            pl.BlockSpec((2, _BR, D), lambda i: (0, i, 0)),
            pl.BlockSpec((_BR, 1), lambda i: (i, 0)),
            pl.BlockSpec((_BR, D), lambda i: (i, 0)),
            pl.BlockSpec((_BR, D), lambda i: (i, 0)),
            pl.BlockSpec((1, 1), full),
            pl.BlockSpec((D, D), full),
            pl.BlockSpec((D, D), full),
            pl.BlockSpec((D, D), full),
            pl.BlockSpec((1, D), full),
            pl.BlockSpec((D, D), full),
            pl.BlockSpec((1, D), full),
            pl.BlockSpec((D, D), full),
            pl.BlockSpec((1, D), full),
        ],
        out_specs=[
            pl.BlockSpec((_BR, D), lambda i: (i, 0)),
            pl.BlockSpec((_BR, D), lambda i: (i, 0)),
        ],
        out_shape=[
            jax.ShapeDtypeStruct((NP_, D), _f32),
            jax.ShapeDtypeStruct((NP_, D), _f32),
        ],
    )(h2_parts, norm, x1, u_pad, lam, w0, w1, w2, bc, ws, bs, wd, bd)


def _tc4_body(op_ref, sp_ref, out_ref):
    sden = sp_ref[0] + sp_ref[1]
    sden = jnp.where(sden > 0.0, sden, 1.0)
    out_ref[...] = (op_ref[0] + op_ref[1]) / sden


def _tc4(out_parts, s_parts):
    return pl.pallas_call(
        _tc4_body,
        grid=(NP_ // _BR,),
        in_specs=[
            pl.BlockSpec((2, _BR, D), lambda i: (0, i, 0)),
            pl.BlockSpec((2, _BR, 1), lambda i: (0, i, 0)),
        ],
        out_specs=pl.BlockSpec((_BR, D), lambda i: (i, 0)),
        out_shape=jax.ShapeDtypeStruct((NP_, D), _f32),
    )(out_parts, s_parts)


def kernel(u, edge_index, lambda_max, W_cheb, b_cheb, W_src, b_src,
           W_dst, b_dst, attn):
    sc_deg, sc_agg, sc_edge_e, sc_softmax_num, sc_wagg = _sc_kernels()

    u_pad = jnp.pad(u, ((0, NP_ - N), (0, 0)))
    pad_e = EP - E
    src = jnp.concatenate([edge_index[0],
                           jnp.full((pad_e,), NP_ - 1, jnp.int32)])
    dst = jnp.concatenate([edge_index[1],
                           jnp.full((pad_e,), NP_ - 1, jnp.int32)])
    src2d = src.reshape(NW, NCH, CH)
    dst2d = dst.reshape(NW, NCH, CH)
    z1 = jnp.zeros((NP_,), _f32)
    zq = jnp.zeros((NP_, _DQ), _f32)
    lam = lambda_max.reshape(1, 1)
    w0 = W_cheb[0 * D:1 * D]
    w1 = W_cheb[1 * D:2 * D]
    w2 = W_cheb[2 * D:3 * D]
    bc = b_cheb.reshape(1, D)
    bs = b_src.reshape(1, D)
    bd = b_dst.reshape(1, D)
    attn_v = attn.reshape(D)

    def _quarters(x):
        return x.reshape(NP_, _NQ, _DQ).transpose(1, 0, 2)

    def _merge(parts):
        return jnp.concatenate([parts[:, q] for q in range(_NQ)], axis=-1)

    deg_parts = sc_deg(dst2d, z1)
    y0, norm, degc = _tc1(deg_parts.reshape(2, NP_, 1), u_pad)
    h1_parts = sc_agg(_quarters(y0), src2d, dst2d, zq)
    x1, y1 = _tc2(_merge(h1_parts), norm, u_pad, lam)
    h2_parts = sc_agg(_quarters(y1), src2d, dst2d, zq)
    fs, fd = _tc3(_merge(h2_parts), norm, x1, u_pad, lam, w0, w1, w2, bc,
                  W_src, bs, W_dst, bd)

    e_edges, se_parts = sc_edge_e(fs, fd, src2d, dst2d, attn_v, z1)
    ee_edges, s_parts = sc_softmax_num(e_edges, dst2d, se_parts,
                                       degc.reshape(NP_), z1)
    out_parts = sc_wagg(_quarters(fs), src2d, dst2d, ee_edges, zq)
    out = _tc4(_merge(out_parts), s_parts.reshape(2, NP_, 1))
    return out[:N]


# speedup vs baseline: 1.8904x; 1.2235x over previous
import functools

import jax
import jax.numpy as jnp
from jax import lax
from jax.experimental import pallas as pl
from jax.experimental.pallas import tpu as pltpu
from jax.experimental.pallas import tpu_sc as plsc

N = 10000
E = 320000
D = 128
NP_ = 10240
NC, NS, L = 2, 16, 16
NW = NC * NS
CPT = D // NW
EPW = 10240
EP = EPW * NW
CH = 128
NCH = EPW // CH
BE = 4096
NBE = EP // BE
RPT = NP_ // NS

_f32 = jnp.float32
_params = pltpu.CompilerParams(use_tc_tiling_on_sc=False,
                               needs_layout_passes=False)


def _wid():
    c = lax.axis_index("c")
    s = lax.axis_index("s")
    return c, s, c * NS + s


@functools.cache
def _sc_kernels():
    mesh = plsc.VectorSubcoreMesh(core_axis_name="c", subcore_axis_name="s",
                                  num_cores=NC, num_subcores=NS)

    def _reduce_tiles(tab_v, red_v, stage_sh, out_ref, c, s):
        pltpu.sync_copy(tab_v, stage_sh.at[s])
        plsc.subcore_barrier()
        for t in range(NS):
            pltpu.sync_copy(stage_sh.at[t, pl.ds(s * RPT, RPT)], red_v)
            if t == 0:
                def cp0(k, carry):
                    tab_v[pl.ds(k * L, L)] = red_v[pl.ds(k * L, L)]
                    return carry
                lax.fori_loop(0, RPT // L, cp0, 0)
            else:
                def acc_t(k, carry):
                    tab_v[pl.ds(k * L, L)] = (tab_v[pl.ds(k * L, L)]
                                              + red_v[pl.ds(k * L, L)])
                    return carry
                lax.fori_loop(0, RPT // L, acc_t, 0)
        pltpu.sync_copy(tab_v.at[pl.ds(0, RPT)],
                        out_ref.at[c, pl.ds(s * RPT, RPT)])

    @functools.partial(
        pl.kernel,
        out_type=jax.ShapeDtypeStruct((NC, NP_), _f32),
        mesh=mesh,
        compiler_params=_params,
        scratch_types=[
            pltpu.VMEM((NCH, CH), jnp.int32),
            pltpu.VMEM((NP_,), _f32),
            pltpu.VMEM((RPT,), _f32),
            pltpu.VMEM_SHARED((NS, NP_), _f32),
        ],
    )
    def _sc_deg(dst_hbm, z1_hbm, deg_out, dst_v, tab_v, red_v, stage_sh):
        c, s, w = _wid()
        pltpu.sync_copy(dst_hbm.at[w], dst_v)
        pltpu.sync_copy(z1_hbm, tab_v)
        ones = jnp.ones((L,), _f32)

        def chunk(j, carry):
            for k in range(CH // L):
                dv = dst_v[j, pl.ds(k * L, L)]
                plsc.addupdate_scatter(tab_v, [dv], ones)
            return carry

        lax.fori_loop(0, NCH, chunk, 0)
        _reduce_tiles(tab_v, red_v, stage_sh, deg_out, c, s)

    DQ = 32
    NQ = D // DQ

    @functools.partial(
        pl.kernel,
        out_type=jax.ShapeDtypeStruct((NC, NQ, NP_, DQ), _f32),
        mesh=mesh,
        compiler_params=_params,
        scratch_types=[
            pltpu.VMEM((NCH, CH), jnp.int32),
            pltpu.VMEM((NCH, CH), jnp.int32),
            pltpu.VMEM((CH, DQ), _f32),
            pltpu.VMEM((CH, DQ), _f32),
            pltpu.VMEM_SHARED((NP_, DQ), _f32),
            pltpu.VMEM_SHARED((NP_, DQ), _f32),
            pltpu.SemaphoreType.DMA,
            pltpu.SemaphoreType.DMA,
        ],
    )
    def _sc_agg(yq_hbm, src_hbm, dst_hbm, zq_hbm, out_hbm,
                src_v, dst_v, buf0, buf1, tab_sh, acc_sh, g0, g1):
        c, s, w = _wid()
        pltpu.sync_copy(src_hbm.at[w], src_v)
        pltpu.sync_copy(dst_hbm.at[w], dst_v)
        for q in range(NQ):
            pltpu.sync_copy(yq_hbm.at[q, pl.ds(s * RPT, RPT)],
                            tab_sh.at[pl.ds(s * RPT, RPT)])
            pltpu.sync_copy(zq_hbm.at[pl.ds(s * RPT, RPT)],
                            acc_sh.at[pl.ds(s * RPT, RPT)])
            plsc.subcore_barrier()
            pltpu.async_copy(tab_sh.at[src_v.at[0]], buf0, g0)
            pltpu.async_copy(tab_sh.at[src_v.at[1]], buf1, g1)

            def pair(jj, carry):
                a = 2 * jj
                b = a + 1
                pltpu.make_async_copy(tab_sh.at[src_v.at[a]],
                                      buf0, g0).wait()
                pltpu.sync_copy(buf0, acc_sh.at[dst_v.at[a]], add=True)

                @pl.when(jj < NCH // 2 - 1)
                def _():
                    pltpu.async_copy(tab_sh.at[src_v.at[a + 2]], buf0, g0)

                pltpu.make_async_copy(tab_sh.at[src_v.at[b]],
                                      buf1, g1).wait()
                pltpu.sync_copy(buf1, acc_sh.at[dst_v.at[b]], add=True)

                @pl.when(jj < NCH // 2 - 1)
                def _():
                    pltpu.async_copy(tab_sh.at[src_v.at[b + 2]], buf1, g1)

                return carry

            lax.fori_loop(0, NCH // 2, pair, 0)
            plsc.subcore_barrier()
            pltpu.sync_copy(acc_sh.at[pl.ds(s * RPT, RPT)],
                            out_hbm.at[c, q, pl.ds(s * RPT, RPT)])

    @functools.partial(
        pl.kernel,
        out_type=(jax.ShapeDtypeStruct((NW, NCH, CH), _f32),
                  jax.ShapeDtypeStruct((NC, NP_), _f32)),
        mesh=mesh,
        compiler_params=_params,
        scratch_types=[
            pltpu.VMEM((NCH, CH), jnp.int32),
            pltpu.VMEM((NCH, CH), jnp.int32),
            pltpu.VMEM((CH, DQ), _f32),
            pltpu.VMEM((CH, DQ), _f32),
            pltpu.VMEM((CH, DQ), _f32),
            pltpu.VMEM((CH, DQ), _f32),
            pltpu.VMEM((NCH, CH), _f32),
            pltpu.VMEM((D,), _f32),
            pltpu.VMEM((NP_,), _f32),
            pltpu.VMEM((RPT,), _f32),
            pltpu.VMEM_SHARED((NP_, DQ), _f32),
            pltpu.VMEM_SHARED((NP_, DQ), _f32),
            pltpu.VMEM_SHARED((NS, NP_), _f32),
            pltpu.SemaphoreType.DMA,
            pltpu.SemaphoreType.DMA,
            pltpu.SemaphoreType.DMA,
            pltpu.SemaphoreType.DMA,
        ],
    )
    def _sc_edge_e(fsq_hbm, fdq_hbm, src_hbm, dst_hbm, attn_hbm, z1_hbm,
                   e_out, se_out,
                   src_v, dst_v, fa0, fa1, fb0, fb1, e_vm, attn_v,
                   tab_v, red_v, tabA_sh, tabB_sh, stage_sh,
                   ga0, ga1, gb0, gb1):
        c, s, w = _wid()
        pltpu.sync_copy(src_hbm.at[w], src_v)
        pltpu.sync_copy(dst_hbm.at[w], dst_v)
        pltpu.sync_copy(attn_hbm, attn_v)
        pltpu.sync_copy(z1_hbm, tab_v)

        lane = lax.iota(jnp.int32, L)

        def zgrp(k, carry):
            j = k // (CH // L)
            g = k % (CH // L)
            e_vm[j, pl.ds(g * L, L)] = jnp.zeros((L,), _f32)
            return carry

        lax.fori_loop(0, NCH * (CH // L), zgrp, 0)

        for q in range(NQ):
            pltpu.sync_copy(fsq_hbm.at[q, pl.ds(s * RPT, RPT)],
                            tabA_sh.at[pl.ds(s * RPT, RPT)])
            pltpu.sync_copy(fdq_hbm.at[q, pl.ds(s * RPT, RPT)],
                            tabB_sh.at[pl.ds(s * RPT, RPT)])
            plsc.subcore_barrier()
            pltpu.async_copy(tabA_sh.at[src_v.at[0]], fa0, ga0)
            pltpu.async_copy(tabB_sh.at[dst_v.at[0]], fb0, gb0)
            pltpu.async_copy(tabA_sh.at[src_v.at[1]], fa1, ga1)
            pltpu.async_copy(tabB_sh.at[dst_v.at[1]], fb1, gb1)

            def compute_chunk(j, fsb, fdb):
                def grp16(g, carry):
                    accs = []
                    for rr in range(L):
                        r = g * L + rr
                        acc = jnp.zeros((L,), _f32)
                        for dg in range(DQ // L):
                            x = (fsb[r, pl.ds(dg * L, L)]
                                 + fdb[r, pl.ds(dg * L, L)])
                            t = jnp.where(x > 0.0, x, 0.2 * x)
                            acc = acc + t * attn_v[pl.ds(q * DQ + dg * L, L)]
                        accs.append(acc)
                    pvec = jnp.zeros((L,), _f32)
                    for rr in range(L):
                        pvec = jnp.where(lane == rr, jnp.sum(accs[rr]), pvec)
                    e_vm[j, pl.ds(g * L, L)] = (
                        e_vm[j, pl.ds(g * L, L)] + pvec)
                    return carry

                lax.fori_loop(0, CH // L, grp16, 0)

            def pair(jj, carry):
                a = 2 * jj
                b = a + 1
                pltpu.make_async_copy(tabA_sh.at[src_v.at[a]],
                                      fa0, ga0).wait()
                pltpu.make_async_copy(tabB_sh.at[dst_v.at[a]],
                                      fb0, gb0).wait()
                compute_chunk(a, fa0, fb0)

                @pl.when(jj < NCH // 2 - 1)
                def _():
                    pltpu.async_copy(tabA_sh.at[src_v.at[a + 2]], fa0, ga0)
                    pltpu.async_copy(tabB_sh.at[dst_v.at[a + 2]], fb0, gb0)

                pltpu.make_async_copy(tabA_sh.at[src_v.at[b]],
                                      fa1, ga1).wait()
                pltpu.make_async_copy(tabB_sh.at[dst_v.at[b]],
                                      fb1, gb1).wait()
                compute_chunk(b, fa1, fb1)

                @pl.when(jj < NCH // 2 - 1)
                def _():
                    pltpu.async_copy(tabA_sh.at[src_v.at[b + 2]], fa1, ga1)
                    pltpu.async_copy(tabB_sh.at[dst_v.at[b + 2]], fb1, gb1)

                return carry

            lax.fori_loop(0, NCH // 2, pair, 0)
            plsc.subcore_barrier()

        def hist(j, carry):
            for k in range(CH // L):
                dv = dst_v[j, pl.ds(k * L, L)]
                plsc.addupdate_scatter(tab_v, [dv],
                                       e_vm[j, pl.ds(k * L, L)])
            return carry

        lax.fori_loop(0, NCH, hist, 0)
        pltpu.sync_copy(e_vm, e_out.at[w])
        _reduce_tiles(tab_v, red_v, stage_sh, se_out, c, s)

    @functools.partial(
        pl.kernel,
        out_type=(jax.ShapeDtypeStruct((NW, NCH, CH), _f32),
                  jax.ShapeDtypeStruct((NC, NP_), _f32)),
        mesh=mesh,
        compiler_params=_params,
        scratch_types=[
            pltpu.VMEM((NCH, CH), jnp.int32),
            pltpu.VMEM((NCH, CH), _f32),
            pltpu.VMEM((NCH, CH), _f32),
            pltpu.VMEM((NP_,), _f32),
            pltpu.VMEM((NP_,), _f32),
            pltpu.VMEM((RPT,), _f32),
            pltpu.VMEM_SHARED((NS, NP_), _f32),
        ],
    )
    def _sc_softmax_num(e_hbm, dst_hbm, se_hbm, degc_hbm, z1_hbm,
                        ee_out, s_out,
                        dst_v, e_vm, ee_vm, b_tab, tab_v, red_v, stage_sh):
        c, s, w = _wid()
        pltpu.sync_copy(dst_hbm.at[w], dst_v)
        pltpu.sync_copy(e_hbm.at[w], e_vm)
        pltpu.sync_copy(se_hbm.at[0], b_tab)
        pltpu.sync_copy(se_hbm.at[1], tab_v)

        def add_grp(k, carry):
            b_tab[pl.ds(k * L, L)] = (b_tab[pl.ds(k * L, L)]
                                      + tab_v[pl.ds(k * L, L)])
            return carry

        lax.fori_loop(0, NP_ // L, add_grp, 0)
        pltpu.sync_copy(degc_hbm, tab_v)

        def div_grp(k, carry):
            b_tab[pl.ds(k * L, L)] = (b_tab[pl.ds(k * L, L)]
                                      / tab_v[pl.ds(k * L, L)])
            return carry

        lax.fori_loop(0, NP_ // L, div_grp, 0)
        pltpu.sync_copy(z1_hbm, tab_v)

        def chunk(j, carry):
            def grp(k, carry2):
                dv = dst_v[j, pl.ds(k * L, L)]
                bv = plsc.load_gather(b_tab, [dv])
                ee = jnp.exp(e_vm[j, pl.ds(k * L, L)] - bv)
                ee_vm[j, pl.ds(k * L, L)] = ee
                plsc.addupdate_scatter(tab_v, [dv], ee)
                return carry2

            lax.fori_loop(0, CH // L, grp, 0)
            return carry

        lax.fori_loop(0, NCH, chunk, 0)
        pltpu.sync_copy(ee_vm, ee_out.at[w])
        _reduce_tiles(tab_v, red_v, stage_sh, s_out, c, s)

    _dn = lax.GatherDimensionNumbers(offset_dims=(),
                                     collapsed_slice_dims=(0,),
                                     start_index_map=(0,))

    @functools.partial(
        pl.kernel,
        out_type=jax.ShapeDtypeStruct((NC, NQ, NP_, DQ), _f32),
        mesh=mesh,
        compiler_params=_params,
        scratch_types=[
            pltpu.VMEM((NCH, CH), jnp.int32),
            pltpu.VMEM((NCH, CH), jnp.int32),
            pltpu.VMEM((NCH, CH), _f32),
            pltpu.VMEM((CH, DQ), _f32),
            pltpu.VMEM((CH, DQ), _f32),
            pltpu.VMEM_SHARED((NP_, DQ), _f32),
            pltpu.VMEM_SHARED((NP_, DQ), _f32),
            pltpu.SemaphoreType.DMA,
            pltpu.SemaphoreType.DMA,
        ],
    )
    def _sc_wagg(fsq_hbm, src_hbm, dst_hbm, ee_hbm, zq_hbm, out_hbm,
                 src_v, dst_v, ee_vm, buf0, buf1, tab_sh, acc_sh, g0, g1):
        c, s, w = _wid()
        pltpu.sync_copy(src_hbm.at[w], src_v)
        pltpu.sync_copy(dst_hbm.at[w], dst_v)
        pltpu.sync_copy(ee_hbm.at[w], ee_vm)

        def scale_chunk(j, buf):
            def grp(g, carry):
                eev = ee_vm[j, pl.ds(g * L, L)]
                for rr in range(L):
                    r = g * L + rr
                    av = lax.gather(
                        eev, jnp.full((L, 1), rr, jnp.int32), _dn, (1,),
                        mode=lax.GatherScatterMode.PROMISE_IN_BOUNDS)
                    for dg in range(DQ // L):
                        buf[r, pl.ds(dg * L, L)] = (
                            buf[r, pl.ds(dg * L, L)] * av)
                return carry

            lax.fori_loop(0, CH // L, grp, 0)

        for q in range(NQ):
            pltpu.sync_copy(fsq_hbm.at[q, pl.ds(s * RPT, RPT)],
                            tab_sh.at[pl.ds(s * RPT, RPT)])
            pltpu.sync_copy(zq_hbm.at[pl.ds(s * RPT, RPT)],
                            acc_sh.at[pl.ds(s * RPT, RPT)])
            plsc.subcore_barrier()
            pltpu.async_copy(tab_sh.at[src_v.at[0]], buf0, g0)
            pltpu.async_copy(tab_sh.at[src_v.at[1]], buf1, g1)

            def pair(jj, carry):
                a = 2 * jj
                b = a + 1
                pltpu.make_async_copy(tab_sh.at[src_v.at[a]],
                                      buf0, g0).wait()
                scale_chunk(a, buf0)
                pltpu.sync_copy(buf0, acc_sh.at[dst_v.at[a]], add=True)

                @pl.when(jj < NCH // 2 - 1)
                def _():
                    pltpu.async_copy(tab_sh.at[src_v.at[a + 2]], buf0, g0)

                pltpu.make_async_copy(tab_sh.at[src_v.at[b]],
                                      buf1, g1).wait()
                scale_chunk(b, buf1)
                pltpu.sync_copy(buf1, acc_sh.at[dst_v.at[b]], add=True)

                @pl.when(jj < NCH // 2 - 1)
                def _():
                    pltpu.async_copy(tab_sh.at[src_v.at[b + 2]], buf1, g1)

                return carry

            lax.fori_loop(0, NCH // 2, pair, 0)
            plsc.subcore_barrier()
            pltpu.sync_copy(acc_sh.at[pl.ds(s * RPT, RPT)],
                            out_hbm.at[c, q, pl.ds(s * RPT, RPT)])

    return _sc_deg, _sc_agg, _sc_edge_e, _sc_softmax_num, _sc_wagg


_BR = 512
_BC = 512
_NQ = 4
_DQ = 32


def _tc1_body(degp_ref, u_ref, y0_ref, norm_ref, degc_ref):
    deg = degp_ref[0] + degp_ref[1]
    degc = jnp.maximum(deg, 1.0)
    norm = lax.rsqrt(degc)
    degc_ref[...] = degc
    norm_ref[...] = norm
    y0_ref[...] = u_ref[...] * norm


def _tc1(deg_parts, u_pad):
    return pl.pallas_call(
        _tc1_body,
        grid=(NP_ // _BR,),
        in_specs=[
            pl.BlockSpec((2, _BR, 1), lambda i: (0, i, 0)),
            pl.BlockSpec((_BR, D), lambda i: (i, 0)),
        ],
        out_specs=[
            pl.BlockSpec((_BR, D), lambda i: (i, 0)),
            pl.BlockSpec((_BR, 1), lambda i: (i, 0)),
            pl.BlockSpec((_BR, 1), lambda i: (i, 0)),
        ],
        out_shape=[
            jax.ShapeDtypeStruct((NP_, D), _f32),
            jax.ShapeDtypeStruct((NP_, 1), _f32),
            jax.ShapeDtypeStruct((NP_, 1), _f32),
        ],
    )(deg_parts, u_pad)


def _tc2_body(h1p_ref, norm_ref, u_ref, lam_ref, x1_ref, y1_ref):
    rn = 2.0 / lam_ref[0, 0]
    h1 = (h1p_ref[0] + h1p_ref[1]) * norm_ref[...]
    x1 = -rn * h1 + u_ref[...] * (rn - 1.0)
    x1_ref[...] = x1
    y1_ref[...] = x1 * norm_ref[...]


def _tc2(h1_parts, norm, u_pad, lam):
    return pl.pallas_call(
        _tc2_body,
        grid=(NP_ // _BR,),
        in_specs=[
            pl.BlockSpec((2, _BR, D), lambda i: (0, i, 0)),
            pl.BlockSpec((_BR, 1), lambda i: (i, 0)),
            pl.BlockSpec((_BR, D), lambda i: (i, 0)),
            pl.BlockSpec((1, 1), lambda i: (0, 0)),
        ],
        out_specs=[
            pl.BlockSpec((_BR, D), lambda i: (i, 0)),
            pl.BlockSpec((_BR, D), lambda i: (i, 0)),
        ],
        out_shape=[
            jax.ShapeDtypeStruct((NP_, D), _f32),
            jax.ShapeDtypeStruct((NP_, D), _f32),
        ],
    )(h1_parts, norm, u_pad, lam)


def _tc3_body(h2p_ref, norm_ref, x1_ref, u_ref, lam_ref,
              w0_ref, w1_ref, w2_ref, bc_ref, ws_ref, bs_ref, wd_ref, bd_ref,
              fs_ref, fd_ref):
    rn = 2.0 / lam_ref[0, 0]
    h2 = (h2p_ref[0] + h2p_ref[1]) * norm_ref[...]
    x1 = x1_ref[...]
    u = u_ref[...]
    x2 = -2.0 * rn * h2 + x1 * (2.0 * rn - 1.0) - u
    h = (jnp.dot(u, w0_ref[...], preferred_element_type=_f32)
         + jnp.dot(x1, w1_ref[...], preferred_element_type=_f32)
         + jnp.dot(x2, w2_ref[...], preferred_element_type=_f32)
         + bc_ref[...])
    h = jnp.maximum(h, 0.0)
    fs_ref[...] = (jnp.dot(h, ws_ref[...], preferred_element_type=_f32)
                   + bs_ref[...])
    fd_ref[...] = (jnp.dot(h, wd_ref[...], preferred_element_type=_f32)
                   + bd_ref[...])


def _tc3(h2_parts, norm, x1, u_pad, lam, w0, w1, w2, bc, ws, bs, wd, bd):
    full = lambda i: (0, 0)
    return pl.pallas_call(
        _tc3_body,
        grid=(NP_ // _BR,),
        in_specs=[
            pl.BlockSpec((2, _BR, D), lambda i: (0, i, 0)),
            pl.BlockSpec((_BR, 1), lambda i: (i, 0)),
            pl.BlockSpec((_BR, D), lambda i: (i, 0)),
            pl.BlockSpec((_BR, D), lambda i: (i, 0)),
            pl.BlockSpec((1, 1), full),
            pl.BlockSpec((D, D), full),
            pl.BlockSpec((D, D), full),
            pl.BlockSpec((D, D), full),
            pl.BlockSpec((1, D), full),
            pl.BlockSpec((D, D), full),
            pl.BlockSpec((1, D), full),
            pl.BlockSpec((D, D), full),
            pl.BlockSpec((1, D), full),
        ],
        out_specs=[
            pl.BlockSpec((_BR, D), lambda i: (i, 0)),
            pl.BlockSpec((_BR, D), lambda i: (i, 0)),
        ],
        out_shape=[
            jax.ShapeDtypeStruct((NP_, D), _f32),
            jax.ShapeDtypeStruct((NP_, D), _f32),
        ],
    )(h2_parts, norm, x1, u_pad, lam, w0, w1, w2, bc, ws, bs, wd, bd)


def _tc4_body(op_ref, sp_ref, out_ref):
    sden = sp_ref[0] + sp_ref[1]
    sden = jnp.where(sden > 0.0, sden, 1.0)
    out_ref[...] = (op_ref[0] + op_ref[1]) / sden


def _tc4(out_parts, s_parts):
    return pl.pallas_call(
        _tc4_body,
        grid=(NP_ // _BR,),
        in_specs=[
            pl.BlockSpec((2, _BR, D), lambda i: (0, i, 0)),
            pl.BlockSpec((2, _BR, 1), lambda i: (0, i, 0)),
        ],
        out_specs=pl.BlockSpec((_BR, D), lambda i: (i, 0)),
        out_shape=jax.ShapeDtypeStruct((NP_, D), _f32),
    )(out_parts, s_parts)


def kernel(u, edge_index, lambda_max, W_cheb, b_cheb, W_src, b_src,
           W_dst, b_dst, attn):
    sc_deg, sc_agg, sc_edge_e, sc_softmax_num, sc_wagg = _sc_kernels()

    u_pad = jnp.pad(u, ((0, NP_ - N), (0, 0)))
    pad_e = EP - E
    src = jnp.concatenate([edge_index[0],
                           jnp.full((pad_e,), NP_ - 1, jnp.int32)])
    dst = jnp.concatenate([edge_index[1],
                           jnp.full((pad_e,), NP_ - 1, jnp.int32)])
    src2d = src.reshape(NW, NCH, CH)
    dst2d = dst.reshape(NW, NCH, CH)
    z1 = jnp.zeros((NP_,), _f32)
    zq = jnp.zeros((NP_, _DQ), _f32)
    lam = lambda_max.reshape(1, 1)
    w0 = W_cheb[0 * D:1 * D]
    w1 = W_cheb[1 * D:2 * D]
    w2 = W_cheb[2 * D:3 * D]
    bc = b_cheb.reshape(1, D)
    bs = b_src.reshape(1, D)
    bd = b_dst.reshape(1, D)
    attn_v = attn.reshape(D)

    def _quarters(x):
        return x.reshape(NP_, _NQ, _DQ).transpose(1, 0, 2)

    def _merge(parts):
        return jnp.concatenate([parts[:, q] for q in range(_NQ)], axis=-1)

    deg_parts = sc_deg(dst2d, z1)
    y0, norm, degc = _tc1(deg_parts.reshape(2, NP_, 1), u_pad)
    h1_parts = sc_agg(_quarters(y0), src2d, dst2d, zq)
    x1, y1 = _tc2(_merge(h1_parts), norm, u_pad, lam)
    h2_parts = sc_agg(_quarters(y1), src2d, dst2d, zq)
    fs, fd = _tc3(_merge(h2_parts), norm, x1, u_pad, lam, w0, w1, w2, bc,
                  W_src, bs, W_dst, bd)

    e_edges, se_parts = sc_edge_e(_quarters(fs), _quarters(fd), src2d,
                                  dst2d, attn_v, z1)
    ee_edges, s_parts = sc_softmax_num(e_edges, dst2d, se_parts,
                                       degc.reshape(NP_), z1)
    out_parts = sc_wagg(_quarters(fs), src2d, dst2d, ee_edges, zq)
    out = _tc4(_merge(out_parts), s_parts.reshape(2, NP_, 1))
    return out[:N]


# speedup vs baseline: 2.1573x; 1.1412x over previous
import functools

import jax
import jax.numpy as jnp
from jax import lax
from jax.experimental import pallas as pl
from jax.experimental.pallas import tpu as pltpu
from jax.experimental.pallas import tpu_sc as plsc

N = 10000
E = 320000
D = 128
NP_ = 10240
NC, NS, L = 2, 16, 16
NW = NC * NS
CPT = D // NW
EPW = 10240
EP = EPW * NW
CH = 128
NCH = EPW // CH
BE = 4096
NBE = EP // BE
RPT = NP_ // NS

_f32 = jnp.float32
_params = pltpu.CompilerParams(use_tc_tiling_on_sc=False,
                               needs_layout_passes=False)


def _wid():
    c = lax.axis_index("c")
    s = lax.axis_index("s")
    return c, s, c * NS + s


@functools.cache
def _sc_kernels():
    mesh = plsc.VectorSubcoreMesh(core_axis_name="c", subcore_axis_name="s",
                                  num_cores=NC, num_subcores=NS)

    def _reduce_tiles(tab_v, red_v, stage_sh, out_ref, c, s):
        pltpu.sync_copy(tab_v, stage_sh.at[s])
        plsc.subcore_barrier()
        for t in range(NS):
            pltpu.sync_copy(stage_sh.at[t, pl.ds(s * RPT, RPT)], red_v)
            if t == 0:
                def cp0(k, carry):
                    tab_v[pl.ds(k * L, L)] = red_v[pl.ds(k * L, L)]
                    return carry
                lax.fori_loop(0, RPT // L, cp0, 0)
            else:
                def acc_t(k, carry):
                    tab_v[pl.ds(k * L, L)] = (tab_v[pl.ds(k * L, L)]
                                              + red_v[pl.ds(k * L, L)])
                    return carry
                lax.fori_loop(0, RPT // L, acc_t, 0)
        pltpu.sync_copy(tab_v.at[pl.ds(0, RPT)],
                        out_ref.at[c, pl.ds(s * RPT, RPT)])

    @functools.partial(
        pl.kernel,
        out_type=jax.ShapeDtypeStruct((NC, NP_), _f32),
        mesh=mesh,
        compiler_params=_params,
        scratch_types=[
            pltpu.VMEM((NCH, CH), jnp.int32),
            pltpu.VMEM((NP_,), _f32),
            pltpu.VMEM((RPT,), _f32),
            pltpu.VMEM_SHARED((NS, NP_), _f32),
        ],
    )
    def _sc_deg(dst_hbm, z1_hbm, deg_out, dst_v, tab_v, red_v, stage_sh):
        c, s, w = _wid()
        pltpu.sync_copy(dst_hbm.at[w], dst_v)
        pltpu.sync_copy(z1_hbm, tab_v)
        ones = jnp.ones((L,), _f32)

        def chunk(j, carry):
            for k in range(CH // L):
                dv = dst_v[j, pl.ds(k * L, L)]
                plsc.addupdate_scatter(tab_v, [dv], ones)
            return carry

        lax.fori_loop(0, NCH, chunk, 0)
        _reduce_tiles(tab_v, red_v, stage_sh, deg_out, c, s)

    DQ = 32
    NQ = D // DQ

    @functools.partial(
        pl.kernel,
        out_type=jax.ShapeDtypeStruct((NC, NQ, NP_, DQ), _f32),
        mesh=mesh,
        compiler_params=_params,
        scratch_types=[
            pltpu.VMEM((NCH, CH), jnp.int32),
            pltpu.VMEM((NCH, CH), jnp.int32),
            pltpu.VMEM((CH, DQ), _f32),
            pltpu.VMEM((CH, DQ), _f32),
            pltpu.VMEM_SHARED((NP_, DQ), _f32),
            pltpu.VMEM_SHARED((NP_, DQ), _f32),
            pltpu.SemaphoreType.DMA,
            pltpu.SemaphoreType.DMA,
        ],
    )
    def _sc_agg(yq_hbm, src_hbm, dst_hbm, zq_hbm, out_hbm,
                src_v, dst_v, buf0, buf1, tab_sh, acc_sh, g0, g1):
        c, s, w = _wid()
        pltpu.sync_copy(src_hbm.at[w], src_v)
        pltpu.sync_copy(dst_hbm.at[w], dst_v)
        for q in range(NQ):
            pltpu.sync_copy(yq_hbm.at[q, pl.ds(s * RPT, RPT)],
                            tab_sh.at[pl.ds(s * RPT, RPT)])
            pltpu.sync_copy(zq_hbm.at[pl.ds(s * RPT, RPT)],
                            acc_sh.at[pl.ds(s * RPT, RPT)])
            plsc.subcore_barrier()
            pltpu.async_copy(tab_sh.at[src_v.at[0]], buf0, g0)
            pltpu.async_copy(tab_sh.at[src_v.at[1]], buf1, g1)

            def pair(jj, carry):
                a = 2 * jj
                b = a + 1
                pltpu.make_async_copy(tab_sh.at[src_v.at[a]],
                                      buf0, g0).wait()
                pltpu.sync_copy(buf0, acc_sh.at[dst_v.at[a]], add=True)

                @pl.when(jj < NCH // 2 - 1)
                def _():
                    pltpu.async_copy(tab_sh.at[src_v.at[a + 2]], buf0, g0)

                pltpu.make_async_copy(tab_sh.at[src_v.at[b]],
                                      buf1, g1).wait()
                pltpu.sync_copy(buf1, acc_sh.at[dst_v.at[b]], add=True)

                @pl.when(jj < NCH // 2 - 1)
                def _():
                    pltpu.async_copy(tab_sh.at[src_v.at[b + 2]], buf1, g1)

                return carry

            lax.fori_loop(0, NCH // 2, pair, 0)
            plsc.subcore_barrier()
            pltpu.sync_copy(acc_sh.at[pl.ds(s * RPT, RPT)],
                            out_hbm.at[c, q, pl.ds(s * RPT, RPT)])

    @functools.partial(
        pl.kernel,
        out_type=(jax.ShapeDtypeStruct((NW, NCH, CH), _f32),
                  jax.ShapeDtypeStruct((NC, NP_), _f32)),
        mesh=mesh,
        compiler_params=_params,
        scratch_types=[
            pltpu.VMEM((NCH, CH), jnp.int32),
            pltpu.VMEM((NCH, CH), jnp.int32),
            pltpu.VMEM((CH, DQ), _f32),
            pltpu.VMEM((CH, DQ), _f32),
            pltpu.VMEM((CH, DQ), _f32),
            pltpu.VMEM((CH, DQ), _f32),
            pltpu.VMEM((NCH, CH), _f32),
            pltpu.VMEM((D,), _f32),
            pltpu.VMEM((NP_,), _f32),
            pltpu.VMEM((RPT,), _f32),
            pltpu.VMEM_SHARED((NP_, DQ), _f32),
            pltpu.VMEM_SHARED((NP_, DQ), _f32),
            pltpu.VMEM_SHARED((NS, NP_), _f32),
            pltpu.SemaphoreType.DMA,
            pltpu.SemaphoreType.DMA,
            pltpu.SemaphoreType.DMA,
            pltpu.SemaphoreType.DMA,
        ],
    )
    def _sc_edge_e(fsq_hbm, fdq_hbm, src_hbm, dst_hbm, attn_hbm, z1_hbm,
                   e_out, se_out,
                   src_v, dst_v, fa0, fa1, fb0, fb1, e_vm, attn_v,
                   tab_v, red_v, tabA_sh, tabB_sh, stage_sh,
                   ga0, ga1, gb0, gb1):
        c, s, w = _wid()
        pltpu.sync_copy(src_hbm.at[w], src_v)
        pltpu.sync_copy(dst_hbm.at[w], dst_v)
        pltpu.sync_copy(attn_hbm, attn_v)
        pltpu.sync_copy(z1_hbm, tab_v)

        lane = lax.iota(jnp.int32, L)

        def zgrp(k, carry):
            j = k // (CH // L)
            g = k % (CH // L)
            e_vm[j, pl.ds(g * L, L)] = jnp.zeros((L,), _f32)
            return carry

        lax.fori_loop(0, NCH * (CH // L), zgrp, 0)

        for q in range(NQ):
            pltpu.sync_copy(fsq_hbm.at[q, pl.ds(s * RPT, RPT)],
                            tabA_sh.at[pl.ds(s * RPT, RPT)])
            pltpu.sync_copy(fdq_hbm.at[q, pl.ds(s * RPT, RPT)],
                            tabB_sh.at[pl.ds(s * RPT, RPT)])
            plsc.subcore_barrier()
            pltpu.async_copy(tabA_sh.at[src_v.at[0]], fa0, ga0)
            pltpu.async_copy(tabB_sh.at[dst_v.at[0]], fb0, gb0)
            pltpu.async_copy(tabA_sh.at[src_v.at[1]], fa1, ga1)
            pltpu.async_copy(tabB_sh.at[dst_v.at[1]], fb1, gb1)

            def compute_chunk(j, fsb, fdb):
                def grp16(g, carry):
                    accs = []
                    for rr in range(L):
                        r = g * L + rr
                        acc = jnp.zeros((L,), _f32)
                        for dg in range(DQ // L):
                            x = (fsb[r, pl.ds(dg * L, L)]
                                 + fdb[r, pl.ds(dg * L, L)])
                            t = jnp.where(x > 0.0, x, 0.2 * x)
                            acc = acc + t * attn_v[pl.ds(q * DQ + dg * L, L)]
                        accs.append(acc)
                    pvec = jnp.zeros((L,), _f32)
                    for rr in range(L):
                        pvec = jnp.where(lane == rr, jnp.sum(accs[rr]), pvec)
                    e_vm[j, pl.ds(g * L, L)] = (
                        e_vm[j, pl.ds(g * L, L)] + pvec)
                    return carry

                lax.fori_loop(0, CH // L, grp16, 0)

            def pair(jj, carry):
                a = 2 * jj
                b = a + 1
                pltpu.make_async_copy(tabA_sh.at[src_v.at[a]],
                                      fa0, ga0).wait()
                pltpu.make_async_copy(tabB_sh.at[dst_v.at[a]],
                                      fb0, gb0).wait()
                compute_chunk(a, fa0, fb0)

                @pl.when(jj < NCH // 2 - 1)
                def _():
                    pltpu.async_copy(tabA_sh.at[src_v.at[a + 2]], fa0, ga0)
                    pltpu.async_copy(tabB_sh.at[dst_v.at[a + 2]], fb0, gb0)

                pltpu.make_async_copy(tabA_sh.at[src_v.at[b]],
                                      fa1, ga1).wait()
                pltpu.make_async_copy(tabB_sh.at[dst_v.at[b]],
                                      fb1, gb1).wait()
                compute_chunk(b, fa1, fb1)

                @pl.when(jj < NCH // 2 - 1)
                def _():
                    pltpu.async_copy(tabA_sh.at[src_v.at[b + 2]], fa1, ga1)
                    pltpu.async_copy(tabB_sh.at[dst_v.at[b + 2]], fb1, gb1)

                return carry

            lax.fori_loop(0, NCH // 2, pair, 0)
            plsc.subcore_barrier()

        def hist(j, carry):
            for k in range(CH // L):
                dv = dst_v[j, pl.ds(k * L, L)]
                plsc.addupdate_scatter(tab_v, [dv],
                                       e_vm[j, pl.ds(k * L, L)])
            return carry

        lax.fori_loop(0, NCH, hist, 0)
        pltpu.sync_copy(e_vm, e_out.at[w])
        _reduce_tiles(tab_v, red_v, stage_sh, se_out, c, s)

    @functools.partial(
        pl.kernel,
        out_type=(jax.ShapeDtypeStruct((NW, NCH, CH), _f32),
                  jax.ShapeDtypeStruct((NC, NP_), _f32)),
        mesh=mesh,
        compiler_params=_params,
        scratch_types=[
            pltpu.VMEM((NCH, CH), jnp.int32),
            pltpu.VMEM((NCH, CH), _f32),
            pltpu.VMEM((NCH, CH), _f32),
            pltpu.VMEM((NP_,), _f32),
            pltpu.VMEM((NP_,), _f32),
            pltpu.VMEM((RPT,), _f32),
            pltpu.VMEM_SHARED((NS, NP_), _f32),
        ],
    )
    def _sc_softmax_num(e_hbm, dst_hbm, se_hbm, degc_hbm, z1_hbm,
                        ee_out, s_out,
                        dst_v, e_vm, ee_vm, b_tab, tab_v, red_v, stage_sh):
        c, s, w = _wid()
        pltpu.sync_copy(dst_hbm.at[w], dst_v)
        pltpu.sync_copy(e_hbm.at[w], e_vm)
        pltpu.sync_copy(se_hbm.at[0], b_tab)
        pltpu.sync_copy(se_hbm.at[1], tab_v)

        def add_grp(k, carry):
            b_tab[pl.ds(k * L, L)] = (b_tab[pl.ds(k * L, L)]
                                      + tab_v[pl.ds(k * L, L)])
            return carry

        lax.fori_loop(0, NP_ // L, add_grp, 0)
        pltpu.sync_copy(degc_hbm, tab_v)

        def div_grp(k, carry):
            b_tab[pl.ds(k * L, L)] = (b_tab[pl.ds(k * L, L)]
                                      / tab_v[pl.ds(k * L, L)])
            return carry

        lax.fori_loop(0, NP_ // L, div_grp, 0)
        pltpu.sync_copy(z1_hbm, tab_v)

        def chunk(j, carry):
            def grp(k, carry2):
                dv = dst_v[j, pl.ds(k * L, L)]
                bv = plsc.load_gather(b_tab, [dv])
                ee = jnp.exp(e_vm[j, pl.ds(k * L, L)] - bv)
                ee_vm[j, pl.ds(k * L, L)] = ee
                plsc.addupdate_scatter(tab_v, [dv], ee)
                return carry2

            lax.fori_loop(0, CH // L, grp, 0)
            return carry

        lax.fori_loop(0, NCH, chunk, 0)
        pltpu.sync_copy(ee_vm, ee_out.at[w])
        _reduce_tiles(tab_v, red_v, stage_sh, s_out, c, s)

    _dn = lax.GatherDimensionNumbers(offset_dims=(),
                                     collapsed_slice_dims=(0,),
                                     start_index_map=(0,))

    @functools.partial(
        pl.kernel,
        out_type=jax.ShapeDtypeStruct((NC, NQ, NP_, DQ), _f32),
        mesh=mesh,
        compiler_params=_params,
        scratch_types=[
            pltpu.VMEM((NCH, CH), jnp.int32),
            pltpu.VMEM((NCH, CH), jnp.int32),
            pltpu.VMEM((NCH, CH), _f32),
            pltpu.VMEM((CH, DQ), _f32),
            pltpu.VMEM((CH, DQ), _f32),
            pltpu.VMEM_SHARED((NP_, DQ), _f32),
            pltpu.VMEM_SHARED((NP_, DQ), _f32),
            pltpu.SemaphoreType.DMA,
            pltpu.SemaphoreType.DMA,
        ],
    )
    def _sc_wagg(fsq_hbm, src_hbm, dst_hbm, ee_hbm, zq_hbm, out_hbm,
                 src_v, dst_v, ee_vm, buf0, buf1, tab_sh, acc_sh, g0, g1):
        c, s, w = _wid()
        pltpu.sync_copy(src_hbm.at[w], src_v)
        pltpu.sync_copy(dst_hbm.at[w], dst_v)
        pltpu.sync_copy(ee_hbm.at[w], ee_vm)

        def scale_chunk(j, buf):
            def grp(g, carry):
                eev = ee_vm[j, pl.ds(g * L, L)]
                for rr in range(L):
                    r = g * L + rr
                    av = lax.gather(
                        eev, jnp.full((L, 1), rr, jnp.int32), _dn, (1,),
                        mode=lax.GatherScatterMode.PROMISE_IN_BOUNDS)
                    for dg in range(DQ // L):
                        buf[r, pl.ds(dg * L, L)] = (
                            buf[r, pl.ds(dg * L, L)] * av)
                return carry

            lax.fori_loop(0, CH // L, grp, 0)

        for q in range(NQ):
            pltpu.sync_copy(fsq_hbm.at[q, pl.ds(s * RPT, RPT)],
                            tab_sh.at[pl.ds(s * RPT, RPT)])
            pltpu.sync_copy(zq_hbm.at[pl.ds(s * RPT, RPT)],
                            acc_sh.at[pl.ds(s * RPT, RPT)])
            plsc.subcore_barrier()
            pltpu.async_copy(tab_sh.at[src_v.at[0]], buf0, g0)
            pltpu.async_copy(tab_sh.at[src_v.at[1]], buf1, g1)

            def pair(jj, carry):
                a = 2 * jj
                b = a + 1
                pltpu.make_async_copy(tab_sh.at[src_v.at[a]],
                                      buf0, g0).wait()
                scale_chunk(a, buf0)
                pltpu.sync_copy(buf0, acc_sh.at[dst_v.at[a]], add=True)

                @pl.when(jj < NCH // 2 - 1)
                def _():
                    pltpu.async_copy(tab_sh.at[src_v.at[a + 2]], buf0, g0)

                pltpu.make_async_copy(tab_sh.at[src_v.at[b]],
                                      buf1, g1).wait()
                scale_chunk(b, buf1)
                pltpu.sync_copy(buf1, acc_sh.at[dst_v.at[b]], add=True)

                @pl.when(jj < NCH // 2 - 1)
                def _():
                    pltpu.async_copy(tab_sh.at[src_v.at[b + 2]], buf1, g1)

                return carry

            lax.fori_loop(0, NCH // 2, pair, 0)
            plsc.subcore_barrier()
            pltpu.sync_copy(acc_sh.at[pl.ds(s * RPT, RPT)],
                            out_hbm.at[c, q, pl.ds(s * RPT, RPT)])

    return _sc_deg, _sc_agg, _sc_edge_e, _sc_softmax_num, _sc_wagg


_BR = 512
_NQ = 4
_DQ = 32


def _split_q(x):
    return [x[:, qq * _DQ:(qq + 1) * _DQ] for qq in range(_NQ)]


def _merge_q(pref):
    p = pref[...]
    hs = p[0] + p[1]
    return jnp.concatenate([hs[qq] for qq in range(_NQ)], axis=-1)


def _tc1_body(degp_ref, u_ref, y0q_ref, norm_ref, degc_ref):
    deg = degp_ref[0] + degp_ref[1]
    degc = jnp.maximum(deg, 1.0)
    norm = lax.rsqrt(degc)
    degc_ref[...] = degc
    norm_ref[...] = norm
    y0 = u_ref[...] * norm
    for qq, ysl in enumerate(_split_q(y0)):
        y0q_ref[qq] = ysl


def _tc1(deg_parts, u_pad):
    return pl.pallas_call(
        _tc1_body,
        grid=(NP_ // _BR,),
        in_specs=[
            pl.BlockSpec((2, _BR, 1), lambda i: (0, i, 0)),
            pl.BlockSpec((_BR, D), lambda i: (i, 0)),
        ],
        out_specs=[
            pl.BlockSpec((_NQ, _BR, _DQ), lambda i: (0, i, 0)),
            pl.BlockSpec((_BR, 1), lambda i: (i, 0)),
            pl.BlockSpec((_BR, 1), lambda i: (i, 0)),
        ],
        out_shape=[
            jax.ShapeDtypeStruct((_NQ, NP_, _DQ), _f32),
            jax.ShapeDtypeStruct((NP_, 1), _f32),
            jax.ShapeDtypeStruct((NP_, 1), _f32),
        ],
    )(deg_parts, u_pad)


def _tc2_body(h1p_ref, norm_ref, u_ref, lam_ref, x1_ref, y1q_ref):
    rn = 2.0 / lam_ref[0, 0]
    h1 = _merge_q(h1p_ref) * norm_ref[...]
    x1 = -rn * h1 + u_ref[...] * (rn - 1.0)
    x1_ref[...] = x1
    y1 = x1 * norm_ref[...]
    for qq, ysl in enumerate(_split_q(y1)):
        y1q_ref[qq] = ysl


def _tc2(h1_parts, norm, u_pad, lam):
    return pl.pallas_call(
        _tc2_body,
        grid=(NP_ // _BR,),
        in_specs=[
            pl.BlockSpec((2, _NQ, _BR, _DQ), lambda i: (0, 0, i, 0)),
            pl.BlockSpec((_BR, 1), lambda i: (i, 0)),
            pl.BlockSpec((_BR, D), lambda i: (i, 0)),
            pl.BlockSpec((1, 1), lambda i: (0, 0)),
        ],
        out_specs=[
            pl.BlockSpec((_BR, D), lambda i: (i, 0)),
            pl.BlockSpec((_NQ, _BR, _DQ), lambda i: (0, i, 0)),
        ],
        out_shape=[
            jax.ShapeDtypeStruct((NP_, D), _f32),
            jax.ShapeDtypeStruct((_NQ, NP_, _DQ), _f32),
        ],
    )(h1_parts, norm, u_pad, lam)


def _tc3_body(h2p_ref, norm_ref, x1_ref, u_ref, lam_ref,
              w0_ref, w1_ref, w2_ref, bc_ref, ws_ref, bs_ref, wd_ref, bd_ref,
              fsq_ref, fdq_ref):
    rn = 2.0 / lam_ref[0, 0]
    h2 = _merge_q(h2p_ref) * norm_ref[...]
    x1 = x1_ref[...]
    u = u_ref[...]
    x2 = -2.0 * rn * h2 + x1 * (2.0 * rn - 1.0) - u
    h = (jnp.dot(u, w0_ref[...], preferred_element_type=_f32)
         + jnp.dot(x1, w1_ref[...], preferred_element_type=_f32)
         + jnp.dot(x2, w2_ref[...], preferred_element_type=_f32)
         + bc_ref[...])
    h = jnp.maximum(h, 0.0)
    fs = jnp.dot(h, ws_ref[...], preferred_element_type=_f32) + bs_ref[...]
    fd = jnp.dot(h, wd_ref[...], preferred_element_type=_f32) + bd_ref[...]
    for qq, fsl in enumerate(_split_q(fs)):
        fsq_ref[qq] = fsl
    for qq, fsl in enumerate(_split_q(fd)):
        fdq_ref[qq] = fsl


def _tc3(h2_parts, norm, x1, u_pad, lam, w0, w1, w2, bc, ws, bs, wd, bd):
    full = lambda i: (0, 0)
    return pl.pallas_call(
        _tc3_body,
        grid=(NP_ // _BR,),
        in_specs=[
            pl.BlockSpec((2, _NQ, _BR, _DQ), lambda i: (0, 0, i, 0)),
            pl.BlockSpec((_BR, 1), lambda i: (i, 0)),
            pl.BlockSpec((_BR, D), lambda i: (i, 0)),
            pl.BlockSpec((_BR, D), lambda i: (i, 0)),
            pl.BlockSpec((1, 1), full),
            pl.BlockSpec((D, D), full),
            pl.BlockSpec((D, D), full),
            pl.BlockSpec((D, D), full),
            pl.BlockSpec((1, D), full),
            pl.BlockSpec((D, D), full),
            pl.BlockSpec((1, D), full),
            pl.BlockSpec((D, D), full),
            pl.BlockSpec((1, D), full),
        ],
        out_specs=[
            pl.BlockSpec((_NQ, _BR, _DQ), lambda i: (0, i, 0)),
            pl.BlockSpec((_NQ, _BR, _DQ), lambda i: (0, i, 0)),
        ],
        out_shape=[
            jax.ShapeDtypeStruct((_NQ, NP_, _DQ), _f32),
            jax.ShapeDtypeStruct((_NQ, NP_, _DQ), _f32),
        ],
    )(h2_parts, norm, x1, u_pad, lam, w0, w1, w2, bc, ws, bs, wd, bd)


def _tc4_body(op_ref, sp_ref, out_ref):
    sden = sp_ref[0] + sp_ref[1]
    sden = jnp.where(sden > 0.0, sden, 1.0)
    out_ref[...] = _merge_q(op_ref) / sden


def _tc4(out_parts, s_parts):
    return pl.pallas_call(
        _tc4_body,
        grid=(NP_ // _BR,),
        in_specs=[
            pl.BlockSpec((2, _NQ, _BR, _DQ), lambda i: (0, 0, i, 0)),
            pl.BlockSpec((2, _BR, 1), lambda i: (0, i, 0)),
        ],
        out_specs=pl.BlockSpec((_BR, D), lambda i: (i, 0)),
        out_shape=jax.ShapeDtypeStruct((NP_, D), _f32),
    )(out_parts, s_parts)


def kernel(u, edge_index, lambda_max, W_cheb, b_cheb, W_src, b_src,
           W_dst, b_dst, attn):
    sc_deg, sc_agg, sc_edge_e, sc_softmax_num, sc_wagg = _sc_kernels()

    u_pad = jnp.pad(u, ((0, NP_ - N), (0, 0)))
    pad_e = EP - E
    src = jnp.concatenate([edge_index[0],
                           jnp.full((pad_e,), NP_ - 1, jnp.int32)])
    dst = jnp.concatenate([edge_index[1],
                           jnp.full((pad_e,), NP_ - 1, jnp.int32)])
    src2d = src.reshape(NW, NCH, CH)
    dst2d = dst.reshape(NW, NCH, CH)
    z1 = jnp.zeros((NP_,), _f32)
    zq = jnp.zeros((NP_, _DQ), _f32)
    lam = lambda_max.reshape(1, 1)
    w0 = W_cheb[0 * D:1 * D]
    w1 = W_cheb[1 * D:2 * D]
    w2 = W_cheb[2 * D:3 * D]
    bc = b_cheb.reshape(1, D)
    bs = b_src.reshape(1, D)
    bd = b_dst.reshape(1, D)
    attn_v = attn.reshape(D)

    deg_parts = sc_deg(dst2d, z1)
    y0q, norm, degc = _tc1(deg_parts.reshape(2, NP_, 1), u_pad)
    h1_parts = sc_agg(y0q, src2d, dst2d, zq)
    x1, y1q = _tc2(h1_parts, norm, u_pad, lam)
    h2_parts = sc_agg(y1q, src2d, dst2d, zq)
    fsq, fdq = _tc3(h2_parts, norm, x1, u_pad, lam, w0, w1, w2, bc,
                    W_src, bs, W_dst, bd)

    e_edges, se_parts = sc_edge_e(fsq, fdq, src2d, dst2d, attn_v, z1)
    ee_edges, s_parts = sc_softmax_num(e_edges, dst2d, se_parts,
                                       degc.reshape(NP_), z1)
    out_parts = sc_wagg(fsq, src2d, dst2d, ee_edges, zq)
    out = _tc4(out_parts, s_parts.reshape(2, NP_, 1))
    return out[:N]
